# baseline (speedup 1.0000x reference)
import functools

import jax
import jax.numpy as jnp
from jax.experimental import pallas as pl
from jax.experimental.pallas import tpu as pltpu
from jax.experimental.pallas import tpu_sc as plsc

_NW = 32


def _sc_gather_rows(table, idx_flat, chunk=1000):
    n = idx_flat.shape[0]
    d = table.shape[1]
    per_w = n // _NW
    n_ch = per_w // chunk
    assert per_w * _NW == n and n_ch * chunk == per_w and chunk % 8 == 0
    mesh = plsc.VectorSubcoreMesh(core_axis_name="c", subcore_axis_name="s")

    @functools.partial(
        pl.kernel, mesh=mesh,
        out_type=jax.ShapeDtypeStruct((n, d), jnp.float32),
        compiler_params=pltpu.CompilerParams(use_tc_tiling_on_sc=False),
        scratch_types=[
            pltpu.VMEM((chunk,), jnp.int32),
            pltpu.VMEM((chunk, d), jnp.float32),
            pltpu.SemaphoreType.DMA,
        ],
    )
    def k(table_hbm, idx_hbm, out_hbm, idx_v, rows_v, sem):
        c = jax.lax.axis_index("c")
        s = jax.lax.axis_index("s")
        wid = s * 2 + c
        base = wid * per_w

        def body(i, carry):
            off = base + i * chunk
            pltpu.sync_copy(idx_hbm.at[pl.ds(off, chunk)], idx_v)
            pltpu.async_copy(table_hbm.at[idx_v], rows_v, sem).wait()
            pltpu.sync_copy(rows_v, out_hbm.at[pl.ds(off, chunk)])
            return carry

        jax.lax.fori_loop(0, n_ch, body, 0)

    return k(table, idx_flat)


def _pick_chunk(per_w, d, budget_rows=1536):
    best = 8
    for ch in range(8, min(per_w, budget_rows) + 1, 8):
        if per_w % ch == 0:
            best = ch
    return best


def _sc_gather(table, idx_flat):
    n = idx_flat.shape[0]
    n_pad = -(-n // (_NW * 8)) * (_NW * 8)
    if n_pad != n:
        idx_flat = jnp.pad(idx_flat, (0, n_pad - n))
    per_w = n_pad // _NW
    out = _sc_gather_rows(table, idx_flat, chunk=_pick_chunk(per_w, table.shape[1]))
    return out[:n] if n_pad != n else out


_QPW = 288
_QPAD = _QPW * _NW
_QHALF = _QPAD // 2


def _sc_gather_sum(pss_flat, fidx_pad, dstq_pad, n_out, p2q):
    d = pss_flat.shape[1]
    k_edges = p2q
    e_per_w = _QPW * k_edges
    chunk_q = _pick_chunk(_QPW, d, budget_rows=max(8, 1536 // k_edges))
    chunk_e = chunk_q * k_edges
    n_ch = _QPW // chunk_q
    mesh = plsc.VectorSubcoreMesh(core_axis_name="c", subcore_axis_name="s")

    @functools.partial(
        pl.kernel, mesh=mesh,
        out_type=jax.ShapeDtypeStruct((_QPAD, d), jnp.float32),
        compiler_params=pltpu.CompilerParams(use_tc_tiling_on_sc=False),
        scratch_types=[
            pltpu.VMEM((chunk_e,), jnp.int32),
            pltpu.VMEM((chunk_e,), jnp.int32),
            pltpu.VMEM((chunk_e, d), jnp.float32),
            pltpu.VMEM((_QPW, d), jnp.float32),
            pltpu.VMEM_SHARED((_QHALF, d), jnp.float32),
            pltpu.SemaphoreType.DMA,
            pltpu.SemaphoreType.DMA,
        ],
    )
    def k(pss_hbm, fidx_hbm, dstq_hbm, out_hbm, idx_v, dst_v, rows_v, zero_v, acc_sh, sem, sem2):
        c = jax.lax.axis_index("c")
        s = jax.lax.axis_index("s")
        qbase_local = s * _QPW
        qbase_glob = c * _QHALF + s * _QPW
        ebase = qbase_glob * k_edges

        def zbody(i, carry):
            for j0 in range(d // 16):
                zero_v[i, j0 * 16:(j0 + 1) * 16] = jnp.zeros((16,), jnp.float32)
            return carry
        jax.lax.fori_loop(0, _QPW, zbody, 0)
        pltpu.sync_copy(zero_v, acc_sh.at[pl.ds(qbase_local, _QPW)])

        def body(i, carry):
            eoff = ebase + i * chunk_e
            pltpu.sync_copy(fidx_hbm.at[pl.ds(eoff, chunk_e)], idx_v)
            pltpu.sync_copy(dstq_hbm.at[pl.ds(eoff, chunk_e)], dst_v)
            pltpu.async_copy(pss_hbm.at[idx_v], rows_v, sem).wait()
            pltpu.async_copy(rows_v, acc_sh.at[dst_v], sem2, add=True).wait()
            return carry
        jax.lax.fori_loop(0, n_ch, body, 0)

        pltpu.sync_copy(acc_sh.at[pl.ds(qbase_local, _QPW)],
                        out_hbm.at[pl.ds(qbase_glob, _QPW)])

    return k(pss_flat, fidx_pad, dstq_pad)[:n_out]

_ZS = {'traffic': [1385.4058837890625, 859.8118896484375], 'packets': [1.4015231132507324, 0.8932565450668335], 'eq_lambda': [1350.97119140625, 858.316162109375], 'avg_pkts_lambda': [0.9117304086685181, 0.9723503589630127], 'exp_max_factor': [6.663637638092041, 4.715115070343018], 'pkts_lambda_on': [0.9116322994232178, 1.651275396347046], 'avg_t_off': [1.6649284362792969, 2.356407403945923], 'avg_t_on': [1.6649284362792969, 2.356407403945923], 'ar_a': [0.0, 1.0], 'sigma': [0.0, 1.0], 'capacity': [27611.091796875, 20090.62109375], 'queue_size': [30259.10546875, 21410.095703125]}
_T = 8
_ITERS = 8


def _lstm(x, h, c, Wx, Wh, b):
    z = x @ Wx + h @ Wh + b
    i, f, g, o = jnp.split(z, 4, axis=-1)
    c = jax.nn.sigmoid(f) * c + jax.nn.sigmoid(i) * jnp.tanh(g)
    h = jax.nn.sigmoid(o) * jnp.tanh(c)
    return h, c


def _rnn(seq, h0, c0, Wx, Wh, b, reverse=False):
    xs = jnp.swapaxes(seq, 0, 1)
    if reverse:
        xs = xs[::-1]
    def step(carry, x):
        h, c = _lstm(x, carry[0], carry[1], Wx, Wh, b)
        return (h, c), h
    (h, c), ys = jax.lax.scan(step, (h0, c0), xs)
    return jnp.swapaxes(ys, 0, 1), h, c


def _mlp2(x, W1, b1, W2, b2):
    return jax.nn.relu(jax.nn.relu(x @ W1 + b1) @ W2 + b2)


_BP = 1000


def _pg(w, hw):
    return jnp.concatenate([w[:, 0:2 * hw], w[:, 3 * hw:4 * hw], w[:, 2 * hw:3 * hw]], axis=1)


def _gates_ifog(z, c, w):
    sig = jax.nn.sigmoid(z[:, 0:3 * w])
    g = jnp.tanh(z[:, 3 * w:4 * w])
    c = sig[:, w:2 * w] * c + sig[:, 0:w] * g
    h = sig[:, 2 * w:3 * w] * jnp.tanh(c)
    return h, c


def _bilstm_body(qg_ref, lg_ref, hf_ref, cf_ref, hb_ref, cb_ref,
                 wqf, wlf, whf, bf, wqb, wlb, whb, bb,
                 pf_ref, pb_ref, hfo, cfo, hbo, cbo):
    b = _BP
    qa = jnp.reshape(qg_ref[...], (_T * b, 32))
    la = jnp.reshape(lg_ref[...], (_T * b, 32))
    zxf = qa @ wqf[...] + la @ wlf[...] + bf[...]
    zxb = qa @ wqb[...] + la @ wlb[...] + bb[...]
    hf = hf_ref[...]
    cf = cf_ref[...]
    hb = hb_ref[...]
    cb = cb_ref[...]
    pf_ref[0] = hf
    pb_ref[0] = hb
    for t in range(_T):
        tb = _T - 1 - t
        zf = zxf[t * b:(t + 1) * b] + hf @ whf[...]
        hf, cf = _gates_ifog(zf, cf, 16)
        zb = zxb[tb * b:(tb + 1) * b] + hb @ whb[...]
        hb, cb = _gates_ifog(zb, cb, 16)
        pf_ref[1 + t] = hf
        pb_ref[1 + tb] = hb
    hfo[...] = hf
    cfo[...] = cf
    hbo[...] = hb
    cbo[...] = cb


def _tc_bilstm(qg, lg, hf, cf, hb, cb, p):
    n_paths = hf.shape[0]
    grid = n_paths // _BP
    b = _BP
    f32 = jnp.float32
    spec_w = lambda r, c: pl.BlockSpec((r, c), lambda i: (0, 0))
    spec_h = pl.BlockSpec((b, 16), lambda i: (i, 0))
    spec_p = pl.BlockSpec((_T + 1, b, 16), lambda i: (0, i, 0))
    out = pl.pallas_call(
        _bilstm_body,
        grid=(grid,),
        in_specs=[
            pl.BlockSpec((_T, b, 32), lambda i: (0, i, 0)),
            pl.BlockSpec((_T, b, 32), lambda i: (0, i, 0)),
            spec_h, spec_h, spec_h, spec_h,
            spec_w(32, 64), spec_w(32, 64), spec_w(16, 64), spec_w(1, 64),
            spec_w(32, 64), spec_w(32, 64), spec_w(16, 64), spec_w(1, 64),
        ],
        out_specs=[spec_p, spec_p, spec_h, spec_h, spec_h, spec_h],
        out_shape=[
            jax.ShapeDtypeStruct((_T + 1, n_paths, 16), f32),
            jax.ShapeDtypeStruct((_T + 1, n_paths, 16), f32),
            jax.ShapeDtypeStruct((n_paths, 16), f32),
            jax.ShapeDtypeStruct((n_paths, 16), f32),
            jax.ShapeDtypeStruct((n_paths, 16), f32),
            jax.ShapeDtypeStruct((n_paths, 16), f32),
        ],
        compiler_params=pltpu.CompilerParams(
            dimension_semantics=("arbitrary",)),
    )(qg, lg, hf, cf, hb, cb,
      _pg(p['pf_Wx'][:32], 16), _pg(p['pf_Wx'][32:], 16), _pg(p['pf_Wh'], 16), _pg(p['pf_b'][None], 16),
      _pg(p['pb_Wx'][:32], 16), _pg(p['pb_Wx'][32:], 16), _pg(p['pb_Wh'], 16), _pg(p['pb_b'][None], 16))
    return out


def _mlp2_body(x_ref, w1, b1, w2, b2, o_ref):
    h = jax.nn.relu(x_ref[...] @ w1[...] + b1[...])
    o_ref[...] = jax.nn.relu(h @ w2[...] + b2[...])


def _tc_mlp2(x, W1, b1, W2, b2, bp=1000):
    n, di = x.shape
    dh, do = W2.shape
    grid = n // bp
    spec_w = lambda r, c: pl.BlockSpec((r, c), lambda i: (0, 0))
    return pl.pallas_call(
        _mlp2_body,
        grid=(grid,),
        in_specs=[
            pl.BlockSpec((bp, di), lambda i: (i, 0)),
            spec_w(di, dh), spec_w(1, dh), spec_w(dh, do), spec_w(1, do),
        ],
        out_specs=pl.BlockSpec((bp, do), lambda i: (i, 0)),
        out_shape=jax.ShapeDtypeStruct((n, do), jnp.float32),
        compiler_params=pltpu.CompilerParams(dimension_semantics=("arbitrary",)),
    )(x, W1, b1[None], W2, b2[None])


def _qlstm_body(sf_ref, sb_ref, h_ref, c_ref, wxf, wxb, wh, b, ho, co):
    z = (sf_ref[...] @ wxf[...] + sb_ref[...] @ wxb[...]
         + h_ref[...] @ wh[...] + b[...])
    h, c = _gates_ifog(z, c_ref[...], 32)
    ho[...] = h
    co[...] = c


def _tc_queue_lstm(sum_f, sum_b, qh, qc, p, bp=1000):
    n = qh.shape[0]
    grid = n // bp
    spec_w = lambda r, c: pl.BlockSpec((r, c), lambda i: (0, 0))
    spec_h = pl.BlockSpec((bp, 32), lambda i: (i, 0))
    spec_s = pl.BlockSpec((bp, 16), lambda i: (i, 0))
    return pl.pallas_call(
        _qlstm_body,
        grid=(grid,),
        in_specs=[spec_s, spec_s, spec_h, spec_h,
                  spec_w(16, 128), spec_w(16, 128), spec_w(32, 128), spec_w(1, 128)],
        out_specs=[spec_h, spec_h],
        out_shape=[jax.ShapeDtypeStruct((n, 32), jnp.float32)] * 2,
        compiler_params=pltpu.CompilerParams(dimension_semantics=("arbitrary",)),
    )(sum_f, sum_b, qh, qc, _pg(p['qu_Wx'][:16], 32), _pg(p['qu_Wx'][16:], 32),
      _pg(p['qu_Wh'], 32), _pg(p['qu_b'][None], 32))


def _lrnn_body(xg_ref, h_ref, c_ref, wx, wh, b, ho, co):
    h = h_ref[...]
    c = c_ref[...]
    for t in range(3):
        z = xg_ref[t] @ wx[...] + h @ wh[...] + b[...]
        h, c = _gates_ifog(z, c, 32)
    ho[...] = h
    co[...] = c


def _tc_link_rnn(qg2, lh, lc, p, bp=1000):
    n = lh.shape[0]
    grid = n // bp
    spec_w = lambda r, c: pl.BlockSpec((r, c), lambda i: (0, 0))
    spec_h = pl.BlockSpec((bp, 32), lambda i: (i, 0))
    return pl.pallas_call(
        _lrnn_body,
        grid=(grid,),
        in_specs=[pl.BlockSpec((3, bp, 32), lambda i: (0, i, 0)),
                  spec_h, spec_h,
                  spec_w(32, 128), spec_w(32, 128), spec_w(1, 128)],
        out_specs=[spec_h, spec_h],
        out_shape=[jax.ShapeDtypeStruct((n, 32), jnp.float32)] * 2,
        compiler_params=pltpu.CompilerParams(dimension_semantics=("arbitrary",)),
    )(qg2, lh, lc, _pg(p['lu_Wx'], 32), _pg(p['lu_Wh'], 32), _pg(p['lu_b'][None], 32))


def _readout_body(pf_ref, pb_ref, icm_ref, w1, b1, w2, b2, w3, b3, qd_ref, ws_ref):
    qd = jnp.zeros_like(icm_ref[0])
    ws = jnp.zeros_like(qd)
    for t in range(_T):
        x = jnp.concatenate([pf_ref[1 + t], pb_ref[1 + t]], axis=1)
        h1 = jax.nn.relu(x @ w1[...] + b1[...])
        h2 = jax.nn.relu(h1 @ w2[...] + b2[...])
        occ = h2 @ w3[...] + b3[...]
        ic = icm_ref[t]
        qd = qd + occ * ic
        ws = ws + ic
    qd_ref[...] = qd
    ws_ref[...] = ws


def _tc_readout(pss_f, pss_b, icm, p, bp=1000):
    n = pss_f.shape[1]
    grid = n // bp
    spec_w = lambda r, c: pl.BlockSpec((r, c), lambda i: (0, 0))
    spec_p = pl.BlockSpec((_T + 1, bp, 16), lambda i: (0, i, 0))
    spec_i = pl.BlockSpec((_T, bp, 16), lambda i: (0, i, 0))
    spec_o = pl.BlockSpec((bp, 16), lambda i: (i, 0))
    return pl.pallas_call(
        _readout_body,
        grid=(grid,),
        in_specs=[spec_p, spec_p, spec_i,
                  spec_w(32, 16), spec_w(1, 16), spec_w(16, 16), spec_w(1, 16),
                  spec_w(16, 1), spec_w(1, 1)],
        out_specs=[spec_o, spec_o],
        out_shape=[jax.ShapeDtypeStruct((n, 16), jnp.float32)] * 2,
        compiler_params=pltpu.CompilerParams(dimension_semantics=("arbitrary",)),
    )(pss_f, pss_b, icm,
      p['ro_W1'], p['ro_b1'][None], p['ro_W2'], p['ro_b2'][None],
      p['ro_W3'], p['ro_b3'][None])


def _scload_body(tr_hbm, idx_hbm, out_hbm, tr_v, idx_v, out_v, sem):
    c = jax.lax.axis_index("c")
    s = jax.lax.axis_index("s")
    wid = s * 2 + c
    n_l = out_v.shape[0]
    n_g = n_l // 16
    pltpu.sync_copy(tr_hbm, tr_v)
    pltpu.sync_copy(idx_hbm.at[pl.ds(wid * n_l * 64, n_l * 64)], idx_v)

    def body(g, carry):
        acc = jnp.zeros((16,), jnp.float32)
        for j in range(64):
            ix = idx_v[pl.ds(g * 1024 + j * 16, 16)]
            acc = acc + plsc.load_gather(tr_v, [ix])
        out_v[pl.ds(g * 16, 16)] = acc
        return carry

    jax.lax.fori_loop(0, n_g, body, 0)
    pltpu.sync_copy(out_v, out_hbm.at[pl.ds(wid * n_l, n_l)])


def _sc_load(traffic_flat, p2l_grp, n_links_pad):
    n_l = n_links_pad // _NW
    mesh = plsc.VectorSubcoreMesh(core_axis_name="c", subcore_axis_name="s")
    k = functools.partial(
        pl.kernel, mesh=mesh,
        out_type=jax.ShapeDtypeStruct((n_links_pad,), jnp.float32),
        compiler_params=pltpu.CompilerParams(use_tc_tiling_on_sc=False,
                                             needs_layout_passes=False),
        scratch_types=[
            pltpu.VMEM(traffic_flat.shape, jnp.float32),
            pltpu.VMEM((n_l * 64,), jnp.int32),
            pltpu.VMEM((n_l,), jnp.float32),
            pltpu.SemaphoreType.DMA,
        ],
    )(_scload_body)
    return k(traffic_flat, p2l_grp)


def kernel(params, traffic, packets, eq_lambda, avg_pkts_lambda, exp_max_factor, pkts_lambda_on, avg_t_off, avg_t_on, ar_a, sigma, capacity, queue_size, weight, length, model, policy, priority, queue_to_path, link_to_path, path_to_link, path_to_queue, queue_to_link):
    p = params
    nz = lambda v, nm: (v - _ZS[nm][0]) / _ZS[nm][1]
    n_paths = queue_to_path.shape[0]
    n_links = capacity.shape[0]
    n_queues = queue_size.shape[0]
    pkt_size = traffic / packets

    nlp = -(-n_links // (_NW * 16)) * (_NW * 16)
    p2l0 = jnp.pad(path_to_link[:, :, 0].astype(jnp.int32), ((0, nlp - n_links), (0, 0)))
    p2l_grp = jnp.swapaxes(p2l0.reshape(nlp // 16, 16, 64), 1, 2).reshape(-1)
    load = (_sc_load(traffic[:, 0], p2l_grp, nlp)[:n_links, None] / capacity)

    path_feat = jnp.concatenate([nz(traffic, 'traffic'), nz(packets, 'packets'), jax.nn.one_hot(model, 7), nz(eq_lambda, 'eq_lambda'), nz(avg_pkts_lambda, 'avg_pkts_lambda'), nz(exp_max_factor, 'exp_max_factor'), nz(pkts_lambda_on, 'pkts_lambda_on'), nz(avg_t_off, 'avg_t_off'), nz(avg_t_on, 'avg_t_on'), nz(ar_a, 'ar_a'), nz(sigma, 'sigma')], axis=1)
    path_state = _tc_mlp2(path_feat, p['pe_W1'], p['pe_b1'], p['pe_W2'], p['pe_b2'])
    h_f = path_state[:, :16]
    c_f = jnp.zeros_like(h_f)
    h_b = path_state[:, 16:]
    c_b = jnp.zeros_like(h_b)
    lh = _tc_mlp2(jnp.concatenate([load, jax.nn.one_hot(policy, 4)], axis=1), p['le_W1'], p['le_b1'], p['le_W2'], p['le_b2'])
    lc = jnp.zeros_like(lh)
    qh = _tc_mlp2(jnp.concatenate([nz(queue_size, 'queue_size'), jax.nn.one_hot(priority, 3), weight], axis=1), p['qe_W1'], p['qe_b1'], p['qe_W2'], p['qe_b2'])
    qc = jnp.zeros_like(qh)

    q2p_tm = queue_to_path.T.reshape(-1).astype(jnp.int32)
    l2p_tm = link_to_path.T.reshape(-1).astype(jnp.int32)
    q2l_tm = queue_to_link.T.reshape(-1).astype(jnp.int32)
    p2q = path_to_queue.shape[1]
    fidx = (path_to_queue[..., 1] * n_paths + path_to_queue[..., 0]).reshape(-1).astype(jnp.int32)
    fidx_pad = jnp.pad(fidx, (0, (_QPAD - n_queues) * p2q))
    dstq_pad = jnp.repeat(jnp.arange(_QPAD, dtype=jnp.int32) % _QHALF, p2q)
    pss_f = pss_b = None
    for _ in range(_ITERS):
        qg = _sc_gather_rows(qh, q2p_tm).reshape(_T, n_paths, 32)
        lg = _sc_gather_rows(lh, l2p_tm).reshape(_T, n_paths, 32)
        pss_f, pss_b, h_f, c_f, h_b, c_b = _tc_bilstm(qg, lg, h_f, c_f, h_b, c_b, p)
        sum_f = _sc_gather_sum(pss_f.reshape((_T + 1) * n_paths, 16), fidx_pad, dstq_pad, n_queues, p2q)
        sum_b = _sc_gather_sum(pss_b.reshape((_T + 1) * n_paths, 16), fidx_pad, dstq_pad, n_queues, p2q)
        qh, qc = _tc_queue_lstm(sum_f[:n_queues], sum_b[:n_queues], qh, qc, p)
        qg2 = _sc_gather(qh, q2l_tm).reshape(3, n_links, 32)
        lh, lc = _tc_link_rnn(qg2, lh, lc, p)

    invtab = jnp.tile(jnp.concatenate([1.0 / capacity, jnp.zeros((1, 1), jnp.float32)]), (1, 16))
    l2p_mask = jnp.where(jnp.arange(_T)[:, None] < length[None, :],
                         link_to_path.T, n_links).reshape(-1).astype(jnp.int32)
    icm = _sc_gather_rows(invtab, l2p_mask).reshape(_T, n_paths, 16)
    qd16, ws16 = _tc_readout(pss_f, pss_b, icm, p)
    return qd16[:, :1] + pkt_size * ws16[:, :1]


# speedup vs baseline: 1.0326x; 1.0326x over previous
import functools

import jax
import jax.numpy as jnp
from jax.experimental import pallas as pl
from jax.experimental.pallas import tpu as pltpu
from jax.experimental.pallas import tpu_sc as plsc

_NW = 32


def _sc_gather_rows(table, idx_flat, chunk=1000):
    n = idx_flat.shape[0]
    d = table.shape[1]
    per_w = n // _NW
    n_ch = per_w // chunk
    assert per_w * _NW == n and n_ch * chunk == per_w and chunk % 8 == 0
    mesh = plsc.VectorSubcoreMesh(core_axis_name="c", subcore_axis_name="s")

    @functools.partial(
        pl.kernel, mesh=mesh,
        out_type=jax.ShapeDtypeStruct((n, d), jnp.float32),
        compiler_params=pltpu.CompilerParams(use_tc_tiling_on_sc=False),
        scratch_types=[
            pltpu.VMEM((chunk,), jnp.int32),
            pltpu.VMEM((chunk, d), jnp.float32),
            pltpu.SemaphoreType.DMA,
        ],
    )
    def k(table_hbm, idx_hbm, out_hbm, idx_v, rows_v, sem):
        c = jax.lax.axis_index("c")
        s = jax.lax.axis_index("s")
        wid = s * 2 + c
        base = wid * per_w

        def body(i, carry):
            off = base + i * chunk
            pltpu.sync_copy(idx_hbm.at[pl.ds(off, chunk)], idx_v)
            pltpu.async_copy(table_hbm.at[idx_v], rows_v, sem).wait()
            pltpu.sync_copy(rows_v, out_hbm.at[pl.ds(off, chunk)])
            return carry

        jax.lax.fori_loop(0, n_ch, body, 0)

    return k(table, idx_flat)


def _sc_gather2(qtab, ltab, qidx, lidx, chunk=1000):
    n = qidx.shape[0]
    d = qtab.shape[1]
    per_w = n // _NW
    n_ch = per_w // chunk
    assert per_w * _NW == n and n_ch * chunk == per_w and chunk % 8 == 0
    mesh = plsc.VectorSubcoreMesh(core_axis_name="c", subcore_axis_name="s")

    @functools.partial(
        pl.kernel, mesh=mesh,
        out_type=[jax.ShapeDtypeStruct((n, d), jnp.float32)] * 2,
        compiler_params=pltpu.CompilerParams(use_tc_tiling_on_sc=False),
        scratch_types=[pltpu.VMEM((chunk,), jnp.int32),
                       pltpu.VMEM((chunk,), jnp.int32),
                       pltpu.VMEM((chunk, d), jnp.float32),
                       pltpu.VMEM((chunk, d), jnp.float32),
                       pltpu.SemaphoreType.DMA, pltpu.SemaphoreType.DMA,
                       pltpu.SemaphoreType.DMA, pltpu.SemaphoreType.DMA],
    )
    def k(qt_hbm, lt_hbm, qi_hbm, li_hbm, qo_hbm, lo_hbm,
          qi, li, qr, lr, sgq, sgl, soq, sol):
        c = jax.lax.axis_index("c")
        s = jax.lax.axis_index("s")
        base = (s * 2 + c) * per_w

        def body(j, carry):
            off = base + j * chunk
            pltpu.sync_copy(qi_hbm.at[pl.ds(off, chunk)], qi)
            pltpu.sync_copy(li_hbm.at[pl.ds(off, chunk)], li)
            gq = pltpu.async_copy(qt_hbm.at[qi], qr, sgq)
            gl = pltpu.async_copy(lt_hbm.at[li], lr, sgl)
            gq.wait()
            oq = pltpu.async_copy(qr, qo_hbm.at[pl.ds(off, chunk)], soq)
            gl.wait()
            ol = pltpu.async_copy(lr, lo_hbm.at[pl.ds(off, chunk)], sol)
            oq.wait()
            ol.wait()
            return carry

        jax.lax.fori_loop(0, n_ch, body, 0)

    return k(qtab, ltab, qidx, lidx)


def _pick_chunk(per_w, d, budget_rows=1536):
    best = 8
    for ch in range(8, min(per_w, budget_rows) + 1, 8):
        if per_w % ch == 0:
            best = ch
    return best


def _sc_gather(table, idx_flat):
    n = idx_flat.shape[0]
    n_pad = -(-n // (_NW * 8)) * (_NW * 8)
    if n_pad != n:
        idx_flat = jnp.pad(idx_flat, (0, n_pad - n))
    per_w = n_pad // _NW
    out = _sc_gather_rows(table, idx_flat, chunk=_pick_chunk(per_w, table.shape[1]))
    return out[:n] if n_pad != n else out


_QPW = 288
_QPAD = _QPW * _NW
_QHALF = _QPAD // 2


def _sc_gather_sum(pss_flat, fidx_pad, dstq_pad, n_out, p2q):
    d = pss_flat.shape[1]
    k_edges = p2q
    e_per_w = _QPW * k_edges
    chunk_q = _pick_chunk(_QPW, d, budget_rows=max(8, 1536 // k_edges))
    chunk_e = chunk_q * k_edges
    n_ch = _QPW // chunk_q
    mesh = plsc.VectorSubcoreMesh(core_axis_name="c", subcore_axis_name="s")

    @functools.partial(
        pl.kernel, mesh=mesh,
        out_type=jax.ShapeDtypeStruct((_QPAD, d), jnp.float32),
        compiler_params=pltpu.CompilerParams(use_tc_tiling_on_sc=False),
        scratch_types=[
            pltpu.VMEM((chunk_e,), jnp.int32),
            pltpu.VMEM((chunk_e,), jnp.int32),
            pltpu.VMEM((chunk_e, d), jnp.float32),
            pltpu.VMEM((_QPW, d), jnp.float32),
            pltpu.VMEM_SHARED((_QHALF, d), jnp.float32),
            pltpu.SemaphoreType.DMA,
            pltpu.SemaphoreType.DMA,
        ],
    )
    def k(pss_hbm, fidx_hbm, dstq_hbm, out_hbm, idx_v, dst_v, rows_v, zero_v, acc_sh, sem, sem2):
        c = jax.lax.axis_index("c")
        s = jax.lax.axis_index("s")
        qbase_local = s * _QPW
        qbase_glob = c * _QHALF + s * _QPW
        ebase = qbase_glob * k_edges

        def zbody(i, carry):
            for j0 in range(d // 16):
                zero_v[i, j0 * 16:(j0 + 1) * 16] = jnp.zeros((16,), jnp.float32)
            return carry
        jax.lax.fori_loop(0, _QPW, zbody, 0)
        pltpu.sync_copy(zero_v, acc_sh.at[pl.ds(qbase_local, _QPW)])

        def body(i, carry):
            eoff = ebase + i * chunk_e
            pltpu.sync_copy(fidx_hbm.at[pl.ds(eoff, chunk_e)], idx_v)
            pltpu.sync_copy(dstq_hbm.at[pl.ds(eoff, chunk_e)], dst_v)
            pltpu.async_copy(pss_hbm.at[idx_v], rows_v, sem).wait()
            pltpu.async_copy(rows_v, acc_sh.at[dst_v], sem2, add=True).wait()
            return carry
        jax.lax.fori_loop(0, n_ch, body, 0)

        pltpu.sync_copy(acc_sh.at[pl.ds(qbase_local, _QPW)],
                        out_hbm.at[pl.ds(qbase_glob, _QPW)])

    return k(pss_flat, fidx_pad, dstq_pad)[:n_out]


def _sc_gsum2(pf_flat, pb_flat, fidx_pad, dstq_pad, n_out, p2q):
    d = pf_flat.shape[1]
    k_edges = p2q
    chunk_q = _pick_chunk(_QPW, d, budget_rows=max(8, 1536 // k_edges))
    chunk_e = chunk_q * k_edges
    n_ch = _QPW // chunk_q
    mesh = plsc.VectorSubcoreMesh(core_axis_name="c", subcore_axis_name="s")

    @functools.partial(
        pl.kernel, mesh=mesh,
        out_type=[jax.ShapeDtypeStruct((_QPAD, d), jnp.float32)] * 2,
        compiler_params=pltpu.CompilerParams(use_tc_tiling_on_sc=False),
        scratch_types=[
            pltpu.VMEM((chunk_e,), jnp.int32),
            pltpu.VMEM((chunk_e,), jnp.int32),
            pltpu.VMEM((chunk_e, d), jnp.float32),
            pltpu.VMEM((chunk_e, d), jnp.float32),
            pltpu.VMEM((_QPW, d), jnp.float32),
            pltpu.VMEM_SHARED((_QHALF, d), jnp.float32),
            pltpu.VMEM_SHARED((_QHALF, d), jnp.float32),
            pltpu.SemaphoreType.DMA, pltpu.SemaphoreType.DMA,
            pltpu.SemaphoreType.DMA, pltpu.SemaphoreType.DMA,
        ],
    )
    def k(pf_hbm, pb_hbm, fidx_hbm, dstq_hbm, of_hbm, ob_hbm,
          idx_v, dst_v, rf_v, rb_v, zero_v, accf, accb, sgf, sgb, saf, sab):
        c = jax.lax.axis_index("c")
        s = jax.lax.axis_index("s")
        qbase_local = s * _QPW
        qbase_glob = c * _QHALF + s * _QPW
        ebase = qbase_glob * k_edges

        def zbody(i, carry):
            for j0 in range(d // 16):
                zero_v[i, j0 * 16:(j0 + 1) * 16] = jnp.zeros((16,), jnp.float32)
            return carry
        jax.lax.fori_loop(0, _QPW, zbody, 0)
        pltpu.sync_copy(zero_v, accf.at[pl.ds(qbase_local, _QPW)])
        pltpu.sync_copy(zero_v, accb.at[pl.ds(qbase_local, _QPW)])

        def body(i, carry):
            eoff = ebase + i * chunk_e
            pltpu.sync_copy(fidx_hbm.at[pl.ds(eoff, chunk_e)], idx_v)
            pltpu.sync_copy(dstq_hbm.at[pl.ds(eoff, chunk_e)], dst_v)
            gf = pltpu.async_copy(pf_hbm.at[idx_v], rf_v, sgf)
            gb = pltpu.async_copy(pb_hbm.at[idx_v], rb_v, sgb)
            gf.wait()
            af = pltpu.async_copy(rf_v, accf.at[dst_v], saf, add=True)
            gb.wait()
            ab = pltpu.async_copy(rb_v, accb.at[dst_v], sab, add=True)
            af.wait()
            ab.wait()
            return carry
        jax.lax.fori_loop(0, n_ch, body, 0)

        pltpu.sync_copy(accf.at[pl.ds(qbase_local, _QPW)],
                        of_hbm.at[pl.ds(qbase_glob, _QPW)])
        pltpu.sync_copy(accb.at[pl.ds(qbase_local, _QPW)],
                        ob_hbm.at[pl.ds(qbase_glob, _QPW)])

    of, ob = k(pf_flat, pb_flat, fidx_pad, dstq_pad)
    return of[:n_out], ob[:n_out]

_ZS = {'traffic': [1385.4058837890625, 859.8118896484375], 'packets': [1.4015231132507324, 0.8932565450668335], 'eq_lambda': [1350.97119140625, 858.316162109375], 'avg_pkts_lambda': [0.9117304086685181, 0.9723503589630127], 'exp_max_factor': [6.663637638092041, 4.715115070343018], 'pkts_lambda_on': [0.9116322994232178, 1.651275396347046], 'avg_t_off': [1.6649284362792969, 2.356407403945923], 'avg_t_on': [1.6649284362792969, 2.356407403945923], 'ar_a': [0.0, 1.0], 'sigma': [0.0, 1.0], 'capacity': [27611.091796875, 20090.62109375], 'queue_size': [30259.10546875, 21410.095703125]}
_T = 8
_ITERS = 8


def _lstm(x, h, c, Wx, Wh, b):
    z = x @ Wx + h @ Wh + b
    i, f, g, o = jnp.split(z, 4, axis=-1)
    c = jax.nn.sigmoid(f) * c + jax.nn.sigmoid(i) * jnp.tanh(g)
    h = jax.nn.sigmoid(o) * jnp.tanh(c)
    return h, c


def _rnn(seq, h0, c0, Wx, Wh, b, reverse=False):
    xs = jnp.swapaxes(seq, 0, 1)
    if reverse:
        xs = xs[::-1]
    def step(carry, x):
        h, c = _lstm(x, carry[0], carry[1], Wx, Wh, b)
        return (h, c), h
    (h, c), ys = jax.lax.scan(step, (h0, c0), xs)
    return jnp.swapaxes(ys, 0, 1), h, c


def _mlp2(x, W1, b1, W2, b2):
    return jax.nn.relu(jax.nn.relu(x @ W1 + b1) @ W2 + b2)


_BP = 1000


def _pg(w, hw):
    return jnp.concatenate([w[:, 0:2 * hw], w[:, 3 * hw:4 * hw], w[:, 2 * hw:3 * hw]], axis=1)


def _gates_ifog(z, c, w):
    sig = jax.nn.sigmoid(z[:, 0:3 * w])
    g = jnp.tanh(z[:, 3 * w:4 * w])
    c = sig[:, w:2 * w] * c + sig[:, 0:w] * g
    h = sig[:, 2 * w:3 * w] * jnp.tanh(c)
    return h, c


def _bilstm_body(qg_ref, lg_ref, hf_ref, cf_ref, hb_ref, cb_ref,
                 wqf, wlf, whf, bf, wqb, wlb, whb, bb,
                 pf_ref, pb_ref, hfo, cfo, hbo, cbo):
    hf = hf_ref[...]
    cf = cf_ref[...]
    hb = hb_ref[...]
    cb = cb_ref[...]
    pf_ref[0] = hf
    pb_ref[0] = hb
    for t in range(_T):
        tb = _T - 1 - t
        zf = qg_ref[t] @ wqf[...] + lg_ref[t] @ wlf[...] + bf[...] + hf @ whf[...]
        hf, cf = _gates_ifog(zf, cf, 16)
        zb = qg_ref[tb] @ wqb[...] + lg_ref[tb] @ wlb[...] + bb[...] + hb @ whb[...]
        hb, cb = _gates_ifog(zb, cb, 16)
        pf_ref[1 + t] = hf
        pb_ref[1 + tb] = hb
    hfo[...] = hf
    cfo[...] = cf
    hbo[...] = hb
    cbo[...] = cb


def _tc_bilstm(qg, lg, hf, cf, hb, cb, p):
    n_paths = hf.shape[0]
    grid = n_paths // _BP
    b = _BP
    f32 = jnp.float32
    spec_w = lambda r, c: pl.BlockSpec((r, c), lambda i: (0, 0))
    spec_h = pl.BlockSpec((b, 16), lambda i: (i, 0))
    spec_p = pl.BlockSpec((_T + 1, b, 16), lambda i: (0, i, 0))
    out = pl.pallas_call(
        _bilstm_body,
        grid=(grid,),
        in_specs=[
            pl.BlockSpec((_T, b, 32), lambda i: (0, i, 0)),
            pl.BlockSpec((_T, b, 32), lambda i: (0, i, 0)),
            spec_h, spec_h, spec_h, spec_h,
            spec_w(32, 64), spec_w(32, 64), spec_w(16, 64), spec_w(1, 64),
            spec_w(32, 64), spec_w(32, 64), spec_w(16, 64), spec_w(1, 64),
        ],
        out_specs=[spec_p, spec_p, spec_h, spec_h, spec_h, spec_h],
        out_shape=[
            jax.ShapeDtypeStruct((_T + 1, n_paths, 16), f32),
            jax.ShapeDtypeStruct((_T + 1, n_paths, 16), f32),
            jax.ShapeDtypeStruct((n_paths, 16), f32),
            jax.ShapeDtypeStruct((n_paths, 16), f32),
            jax.ShapeDtypeStruct((n_paths, 16), f32),
            jax.ShapeDtypeStruct((n_paths, 16), f32),
        ],
        compiler_params=pltpu.CompilerParams(
            dimension_semantics=("arbitrary",)),
    )(qg, lg, hf, cf, hb, cb,
      _pg(p['pf_Wx'][:32], 16), _pg(p['pf_Wx'][32:], 16), _pg(p['pf_Wh'], 16), _pg(p['pf_b'][None], 16),
      _pg(p['pb_Wx'][:32], 16), _pg(p['pb_Wx'][32:], 16), _pg(p['pb_Wh'], 16), _pg(p['pb_b'][None], 16))
    return out


def _mlp2_body(x_ref, w1, b1, w2, b2, o_ref):
    h = jax.nn.relu(x_ref[...] @ w1[...] + b1[...])
    o_ref[...] = jax.nn.relu(h @ w2[...] + b2[...])


def _tc_mlp2(x, W1, b1, W2, b2, bp=1000):
    n, di = x.shape
    dh, do = W2.shape
    grid = n // bp
    spec_w = lambda r, c: pl.BlockSpec((r, c), lambda i: (0, 0))
    return pl.pallas_call(
        _mlp2_body,
        grid=(grid,),
        in_specs=[
            pl.BlockSpec((bp, di), lambda i: (i, 0)),
            spec_w(di, dh), spec_w(1, dh), spec_w(dh, do), spec_w(1, do),
        ],
        out_specs=pl.BlockSpec((bp, do), lambda i: (i, 0)),
        out_shape=jax.ShapeDtypeStruct((n, do), jnp.float32),
        compiler_params=pltpu.CompilerParams(dimension_semantics=("arbitrary",)),
    )(x, W1, b1[None], W2, b2[None])


def _qlstm_body(sf_ref, sb_ref, h_ref, c_ref, wxf, wxb, wh, b, ho, co):
    z = (sf_ref[...] @ wxf[...] + sb_ref[...] @ wxb[...]
         + h_ref[...] @ wh[...] + b[...])
    h, c = _gates_ifog(z, c_ref[...], 32)
    ho[...] = h
    co[...] = c


def _tc_queue_lstm(sum_f, sum_b, qh, qc, p, bp=1000):
    n = qh.shape[0]
    grid = n // bp
    spec_w = lambda r, c: pl.BlockSpec((r, c), lambda i: (0, 0))
    spec_h = pl.BlockSpec((bp, 32), lambda i: (i, 0))
    spec_s = pl.BlockSpec((bp, 16), lambda i: (i, 0))
    return pl.pallas_call(
        _qlstm_body,
        grid=(grid,),
        in_specs=[spec_s, spec_s, spec_h, spec_h,
                  spec_w(16, 128), spec_w(16, 128), spec_w(32, 128), spec_w(1, 128)],
        out_specs=[spec_h, spec_h],
        out_shape=[jax.ShapeDtypeStruct((n, 32), jnp.float32)] * 2,
        compiler_params=pltpu.CompilerParams(dimension_semantics=("arbitrary",)),
    )(sum_f, sum_b, qh, qc, _pg(p['qu_Wx'][:16], 32), _pg(p['qu_Wx'][16:], 32),
      _pg(p['qu_Wh'], 32), _pg(p['qu_b'][None], 32))


def _lrnn_body(xg_ref, h_ref, c_ref, wx, wh, b, ho, co):
    h = h_ref[...]
    c = c_ref[...]
    for t in range(3):
        z = xg_ref[t] @ wx[...] + h @ wh[...] + b[...]
        h, c = _gates_ifog(z, c, 32)
    ho[...] = h
    co[...] = c


def _tc_link_rnn(qg2, lh, lc, p, bp=1000):
    n = lh.shape[0]
    grid = n // bp
    spec_w = lambda r, c: pl.BlockSpec((r, c), lambda i: (0, 0))
    spec_h = pl.BlockSpec((bp, 32), lambda i: (i, 0))
    return pl.pallas_call(
        _lrnn_body,
        grid=(grid,),
        in_specs=[pl.BlockSpec((3, bp, 32), lambda i: (0, i, 0)),
                  spec_h, spec_h,
                  spec_w(32, 128), spec_w(32, 128), spec_w(1, 128)],
        out_specs=[spec_h, spec_h],
        out_shape=[jax.ShapeDtypeStruct((n, 32), jnp.float32)] * 2,
        compiler_params=pltpu.CompilerParams(dimension_semantics=("arbitrary",)),
    )(qg2, lh, lc, _pg(p['lu_Wx'], 32), _pg(p['lu_Wh'], 32), _pg(p['lu_b'][None], 32))


def _readout_body(pf_ref, pb_ref, icm_ref, w1, b1, w2, b2, w3, b3, qd_ref, ws_ref):
    qd = jnp.zeros_like(icm_ref[0])
    ws = jnp.zeros_like(qd)
    for t in range(_T):
        x = jnp.concatenate([pf_ref[1 + t], pb_ref[1 + t]], axis=1)
        h1 = jax.nn.relu(x @ w1[...] + b1[...])
        h2 = jax.nn.relu(h1 @ w2[...] + b2[...])
        occ = h2 @ w3[...] + b3[...]
        ic = icm_ref[t]
        qd = qd + occ * ic
        ws = ws + ic
    qd_ref[...] = qd
    ws_ref[...] = ws


def _tc_readout(pss_f, pss_b, icm, p, bp=1000):
    n = pss_f.shape[1]
    grid = n // bp
    spec_w = lambda r, c: pl.BlockSpec((r, c), lambda i: (0, 0))
    spec_p = pl.BlockSpec((_T + 1, bp, 16), lambda i: (0, i, 0))
    spec_i = pl.BlockSpec((_T, bp, 16), lambda i: (0, i, 0))
    spec_o = pl.BlockSpec((bp, 16), lambda i: (i, 0))
    return pl.pallas_call(
        _readout_body,
        grid=(grid,),
        in_specs=[spec_p, spec_p, spec_i,
                  spec_w(32, 16), spec_w(1, 16), spec_w(16, 16), spec_w(1, 16),
                  spec_w(16, 1), spec_w(1, 1)],
        out_specs=[spec_o, spec_o],
        out_shape=[jax.ShapeDtypeStruct((n, 16), jnp.float32)] * 2,
        compiler_params=pltpu.CompilerParams(dimension_semantics=("arbitrary",)),
    )(pss_f, pss_b, icm,
      p['ro_W1'], p['ro_b1'][None], p['ro_W2'], p['ro_b2'][None],
      p['ro_W3'], p['ro_b3'][None])


def _scload_body(tr_hbm, idx_hbm, out_hbm, tr_v, idx_v, out_v, sem):
    c = jax.lax.axis_index("c")
    s = jax.lax.axis_index("s")
    wid = s * 2 + c
    n_l = out_v.shape[0]
    n_g = n_l // 16
    pltpu.sync_copy(tr_hbm, tr_v)
    pltpu.sync_copy(idx_hbm.at[pl.ds(wid * n_l * 64, n_l * 64)], idx_v)

    def body(g, carry):
        acc = jnp.zeros((16,), jnp.float32)
        for j in range(64):
            ix = idx_v[pl.ds(g * 1024 + j * 16, 16)]
            acc = acc + plsc.load_gather(tr_v, [ix])
        out_v[pl.ds(g * 16, 16)] = acc
        return carry

    jax.lax.fori_loop(0, n_g, body, 0)
    pltpu.sync_copy(out_v, out_hbm.at[pl.ds(wid * n_l, n_l)])


def _sc_load(traffic_flat, p2l_grp, n_links_pad):
    n_l = n_links_pad // _NW
    mesh = plsc.VectorSubcoreMesh(core_axis_name="c", subcore_axis_name="s")
    k = functools.partial(
        pl.kernel, mesh=mesh,
        out_type=jax.ShapeDtypeStruct((n_links_pad,), jnp.float32),
        compiler_params=pltpu.CompilerParams(use_tc_tiling_on_sc=False,
                                             needs_layout_passes=False),
        scratch_types=[
            pltpu.VMEM(traffic_flat.shape, jnp.float32),
            pltpu.VMEM((n_l * 64,), jnp.int32),
            pltpu.VMEM((n_l,), jnp.float32),
            pltpu.SemaphoreType.DMA,
        ],
    )(_scload_body)
    return k(traffic_flat, p2l_grp)


def kernel(params, traffic, packets, eq_lambda, avg_pkts_lambda, exp_max_factor, pkts_lambda_on, avg_t_off, avg_t_on, ar_a, sigma, capacity, queue_size, weight, length, model, policy, priority, queue_to_path, link_to_path, path_to_link, path_to_queue, queue_to_link):
    p = params
    nz = lambda v, nm: (v - _ZS[nm][0]) / _ZS[nm][1]
    n_paths = queue_to_path.shape[0]
    n_links = capacity.shape[0]
    n_queues = queue_size.shape[0]
    pkt_size = traffic / packets

    nlp = -(-n_links // (_NW * 16)) * (_NW * 16)
    p2l0 = jnp.pad(path_to_link[:, :, 0].astype(jnp.int32), ((0, nlp - n_links), (0, 0)))
    p2l_grp = jnp.swapaxes(p2l0.reshape(nlp // 16, 16, 64), 1, 2).reshape(-1)
    load = (_sc_load(traffic[:, 0], p2l_grp, nlp)[:n_links, None] / capacity)

    path_feat = jnp.concatenate([nz(traffic, 'traffic'), nz(packets, 'packets'), jax.nn.one_hot(model, 7), nz(eq_lambda, 'eq_lambda'), nz(avg_pkts_lambda, 'avg_pkts_lambda'), nz(exp_max_factor, 'exp_max_factor'), nz(pkts_lambda_on, 'pkts_lambda_on'), nz(avg_t_off, 'avg_t_off'), nz(avg_t_on, 'avg_t_on'), nz(ar_a, 'ar_a'), nz(sigma, 'sigma')], axis=1)
    path_state = _tc_mlp2(path_feat, p['pe_W1'], p['pe_b1'], p['pe_W2'], p['pe_b2'])
    h_f = path_state[:, :16]
    c_f = jnp.zeros_like(h_f)
    h_b = path_state[:, 16:]
    c_b = jnp.zeros_like(h_b)
    lh = _tc_mlp2(jnp.concatenate([load, jax.nn.one_hot(policy, 4)], axis=1), p['le_W1'], p['le_b1'], p['le_W2'], p['le_b2'])
    lc = jnp.zeros_like(lh)
    qh = _tc_mlp2(jnp.concatenate([nz(queue_size, 'queue_size'), jax.nn.one_hot(priority, 3), weight], axis=1), p['qe_W1'], p['qe_b1'], p['qe_W2'], p['qe_b2'])
    qc = jnp.zeros_like(qh)

    q2p_tm = queue_to_path.T.reshape(-1).astype(jnp.int32)
    l2p_tm = link_to_path.T.reshape(-1).astype(jnp.int32)
    q2l_tm = queue_to_link.T.reshape(-1).astype(jnp.int32)
    p2q = path_to_queue.shape[1]
    fidx = (path_to_queue[..., 1] * n_paths + path_to_queue[..., 0]).reshape(-1).astype(jnp.int32)
    fidx_pad = jnp.pad(fidx, (0, (_QPAD - n_queues) * p2q))
    dstq_pad = jnp.repeat(jnp.arange(_QPAD, dtype=jnp.int32) % _QHALF, p2q)
    pss_f = pss_b = None
    for _ in range(_ITERS):
        qg, lg = _sc_gather2(qh, lh, q2p_tm, l2p_tm)
        qg = qg.reshape(_T, n_paths, 32)
        lg = lg.reshape(_T, n_paths, 32)
        pss_f, pss_b, h_f, c_f, h_b, c_b = _tc_bilstm(qg, lg, h_f, c_f, h_b, c_b, p)
        sum_f, sum_b = _sc_gsum2(pss_f.reshape((_T + 1) * n_paths, 16),
                                 pss_b.reshape((_T + 1) * n_paths, 16),
                                 fidx_pad, dstq_pad, n_queues, p2q)
        qh, qc = _tc_queue_lstm(sum_f, sum_b, qh, qc, p)
        qg2 = _sc_gather(qh, q2l_tm).reshape(3, n_links, 32)
        lh, lc = _tc_link_rnn(qg2, lh, lc, p)

    invtab = jnp.tile(jnp.concatenate([1.0 / capacity, jnp.zeros((1, 1), jnp.float32)]), (1, 16))
    l2p_mask = jnp.where(jnp.arange(_T)[:, None] < length[None, :],
                         link_to_path.T, n_links).reshape(-1).astype(jnp.int32)
    icm = _sc_gather_rows(invtab, l2p_mask).reshape(_T, n_paths, 16)
    qd16, ws16 = _tc_readout(pss_f, pss_b, icm, p)
    return qd16[:, :1] + pkt_size * ws16[:, :1]


# speedup vs baseline: 1.0828x; 1.0487x over previous
import functools

import jax
import jax.numpy as jnp
from jax.experimental import pallas as pl
from jax.experimental.pallas import tpu as pltpu
from jax.experimental.pallas import tpu_sc as plsc

_NW = 32


def _sc_gather_rows(table, idx_flat, chunk=1000):
    n = idx_flat.shape[0]
    d = table.shape[1]
    per_w = n // _NW
    n_ch = per_w // chunk
    assert per_w * _NW == n and n_ch * chunk == per_w and chunk % 8 == 0
    mesh = plsc.VectorSubcoreMesh(core_axis_name="c", subcore_axis_name="s")

    @functools.partial(
        pl.kernel, mesh=mesh,
        out_type=jax.ShapeDtypeStruct((n, d), jnp.float32),
        compiler_params=pltpu.CompilerParams(use_tc_tiling_on_sc=False),
        scratch_types=[
            pltpu.VMEM((chunk,), jnp.int32),
            pltpu.VMEM((chunk, d), jnp.float32),
            pltpu.SemaphoreType.DMA,
        ],
    )
    def k(table_hbm, idx_hbm, out_hbm, idx_v, rows_v, sem):
        c = jax.lax.axis_index("c")
        s = jax.lax.axis_index("s")
        wid = s * 2 + c
        base = wid * per_w

        def body(i, carry):
            off = base + i * chunk
            pltpu.sync_copy(idx_hbm.at[pl.ds(off, chunk)], idx_v)
            pltpu.async_copy(table_hbm.at[idx_v], rows_v, sem).wait()
            pltpu.sync_copy(rows_v, out_hbm.at[pl.ds(off, chunk)])
            return carry

        jax.lax.fori_loop(0, n_ch, body, 0)

    return k(table, idx_flat)


def _sc_gather2(qtab, ltab, qidx, lidx, chunk=1000):
    n = qidx.shape[0]
    d = qtab.shape[1]
    per_w = n // _NW
    n_ch = per_w // chunk
    assert per_w * _NW == n and n_ch * chunk == per_w and chunk % 8 == 0
    mesh = plsc.VectorSubcoreMesh(core_axis_name="c", subcore_axis_name="s")

    @functools.partial(
        pl.kernel, mesh=mesh,
        out_type=[jax.ShapeDtypeStruct((n, d), jnp.float32)] * 2,
        compiler_params=pltpu.CompilerParams(use_tc_tiling_on_sc=False),
        scratch_types=[pltpu.VMEM((chunk,), jnp.int32),
                       pltpu.VMEM((chunk,), jnp.int32),
                       pltpu.VMEM((chunk, d), jnp.float32),
                       pltpu.VMEM((chunk, d), jnp.float32),
                       pltpu.SemaphoreType.DMA, pltpu.SemaphoreType.DMA,
                       pltpu.SemaphoreType.DMA, pltpu.SemaphoreType.DMA],
    )
    def k(qt_hbm, lt_hbm, qi_hbm, li_hbm, qo_hbm, lo_hbm,
          qi, li, qr, lr, sgq, sgl, soq, sol):
        c = jax.lax.axis_index("c")
        s = jax.lax.axis_index("s")
        base = (s * 2 + c) * per_w

        def body(j, carry):
            off = base + j * chunk
            pltpu.sync_copy(qi_hbm.at[pl.ds(off, chunk)], qi)
            pltpu.sync_copy(li_hbm.at[pl.ds(off, chunk)], li)
            gq = pltpu.async_copy(qt_hbm.at[qi], qr, sgq)
            gl = pltpu.async_copy(lt_hbm.at[li], lr, sgl)
            gq.wait()
            oq = pltpu.async_copy(qr, qo_hbm.at[pl.ds(off, chunk)], soq)
            gl.wait()
            ol = pltpu.async_copy(lr, lo_hbm.at[pl.ds(off, chunk)], sol)
            oq.wait()
            ol.wait()
            return carry

        jax.lax.fori_loop(0, n_ch, body, 0)

    return k(qtab, ltab, qidx, lidx)


def _pick_chunk(per_w, d, budget_rows=1536):
    best = 8
    for ch in range(8, min(per_w, budget_rows) + 1, 8):
        if per_w % ch == 0:
            best = ch
    return best


def _sc_gather(table, idx_flat):
    n = idx_flat.shape[0]
    n_pad = -(-n // (_NW * 8)) * (_NW * 8)
    if n_pad != n:
        idx_flat = jnp.pad(idx_flat, (0, n_pad - n))
    per_w = n_pad // _NW
    out = _sc_gather_rows(table, idx_flat, chunk=_pick_chunk(per_w, table.shape[1]))
    return out[:n] if n_pad != n else out


_QPW = 288
_QPAD = _QPW * _NW
_QHALF = _QPAD // 2


def _sc_gather_sum(pss_flat, fidx_pad, dstq_pad, n_out, p2q):
    d = pss_flat.shape[1]
    k_edges = p2q
    e_per_w = _QPW * k_edges
    chunk_q = _pick_chunk(_QPW, d, budget_rows=max(8, 1536 // k_edges))
    chunk_e = chunk_q * k_edges
    n_ch = _QPW // chunk_q
    mesh = plsc.VectorSubcoreMesh(core_axis_name="c", subcore_axis_name="s")

    @functools.partial(
        pl.kernel, mesh=mesh,
        out_type=jax.ShapeDtypeStruct((_QPAD, d), jnp.float32),
        compiler_params=pltpu.CompilerParams(use_tc_tiling_on_sc=False),
        scratch_types=[
            pltpu.VMEM((chunk_e,), jnp.int32),
            pltpu.VMEM((chunk_e,), jnp.int32),
            pltpu.VMEM((chunk_e, d), jnp.float32),
            pltpu.VMEM((_QPW, d), jnp.float32),
            pltpu.VMEM_SHARED((_QHALF, d), jnp.float32),
            pltpu.SemaphoreType.DMA,
            pltpu.SemaphoreType.DMA,
        ],
    )
    def k(pss_hbm, fidx_hbm, dstq_hbm, out_hbm, idx_v, dst_v, rows_v, zero_v, acc_sh, sem, sem2):
        c = jax.lax.axis_index("c")
        s = jax.lax.axis_index("s")
        qbase_local = s * _QPW
        qbase_glob = c * _QHALF + s * _QPW
        ebase = qbase_glob * k_edges

        def zbody(i, carry):
            for j0 in range(d // 16):
                zero_v[i, j0 * 16:(j0 + 1) * 16] = jnp.zeros((16,), jnp.float32)
            return carry
        jax.lax.fori_loop(0, _QPW, zbody, 0)
        pltpu.sync_copy(zero_v, acc_sh.at[pl.ds(qbase_local, _QPW)])

        def body(i, carry):
            eoff = ebase + i * chunk_e
            pltpu.sync_copy(fidx_hbm.at[pl.ds(eoff, chunk_e)], idx_v)
            pltpu.sync_copy(dstq_hbm.at[pl.ds(eoff, chunk_e)], dst_v)
            pltpu.async_copy(pss_hbm.at[idx_v], rows_v, sem).wait()
            pltpu.async_copy(rows_v, acc_sh.at[dst_v], sem2, add=True).wait()
            return carry
        jax.lax.fori_loop(0, n_ch, body, 0)

        pltpu.sync_copy(acc_sh.at[pl.ds(qbase_local, _QPW)],
                        out_hbm.at[pl.ds(qbase_glob, _QPW)])

    return k(pss_flat, fidx_pad, dstq_pad)[:n_out]


def _sc_gsum2(pf_flat, pb_flat, fidx_pad, dstq_pad, n_out, p2q):
    d = pf_flat.shape[1]
    k_edges = p2q
    chunk_q = _pick_chunk(_QPW, d, budget_rows=max(8, 1536 // k_edges))
    chunk_e = chunk_q * k_edges
    n_ch = _QPW // chunk_q
    mesh = plsc.VectorSubcoreMesh(core_axis_name="c", subcore_axis_name="s")

    @functools.partial(
        pl.kernel, mesh=mesh,
        out_type=[jax.ShapeDtypeStruct((_QPAD, d), jnp.float32)] * 2,
        compiler_params=pltpu.CompilerParams(use_tc_tiling_on_sc=False),
        scratch_types=[
            pltpu.VMEM((chunk_e,), jnp.int32),
            pltpu.VMEM((chunk_e,), jnp.int32),
            pltpu.VMEM((chunk_e, d), jnp.float32),
            pltpu.VMEM((chunk_e, d), jnp.float32),
            pltpu.VMEM((_QPW, d), jnp.float32),
            pltpu.VMEM_SHARED((_QHALF, d), jnp.float32),
            pltpu.VMEM_SHARED((_QHALF, d), jnp.float32),
            pltpu.SemaphoreType.DMA, pltpu.SemaphoreType.DMA,
            pltpu.SemaphoreType.DMA, pltpu.SemaphoreType.DMA,
        ],
    )
    def k(pf_hbm, pb_hbm, fidx_hbm, dstq_hbm, of_hbm, ob_hbm,
          idx_v, dst_v, rf_v, rb_v, zero_v, accf, accb, sgf, sgb, saf, sab):
        c = jax.lax.axis_index("c")
        s = jax.lax.axis_index("s")
        qbase_local = s * _QPW
        qbase_glob = c * _QHALF + s * _QPW
        ebase = qbase_glob * k_edges

        def zbody(i, carry):
            for j0 in range(d // 16):
                zero_v[i, j0 * 16:(j0 + 1) * 16] = jnp.zeros((16,), jnp.float32)
            return carry
        jax.lax.fori_loop(0, _QPW, zbody, 0)
        pltpu.sync_copy(zero_v, accf.at[pl.ds(qbase_local, _QPW)])
        pltpu.sync_copy(zero_v, accb.at[pl.ds(qbase_local, _QPW)])

        def body(i, carry):
            eoff = ebase + i * chunk_e
            pltpu.sync_copy(fidx_hbm.at[pl.ds(eoff, chunk_e)], idx_v)
            pltpu.sync_copy(dstq_hbm.at[pl.ds(eoff, chunk_e)], dst_v)
            gf = pltpu.async_copy(pf_hbm.at[idx_v], rf_v, sgf)
            gb = pltpu.async_copy(pb_hbm.at[idx_v], rb_v, sgb)
            gf.wait()
            af = pltpu.async_copy(rf_v, accf.at[dst_v], saf, add=True)
            gb.wait()
            ab = pltpu.async_copy(rb_v, accb.at[dst_v], sab, add=True)
            af.wait()
            ab.wait()
            return carry
        jax.lax.fori_loop(0, n_ch, body, 0)

        pltpu.sync_copy(accf.at[pl.ds(qbase_local, _QPW)],
                        of_hbm.at[pl.ds(qbase_glob, _QPW)])
        pltpu.sync_copy(accb.at[pl.ds(qbase_local, _QPW)],
                        ob_hbm.at[pl.ds(qbase_glob, _QPW)])

    of, ob = k(pf_flat, pb_flat, fidx_pad, dstq_pad)
    return of[:n_out], ob[:n_out]

_ZS = {'traffic': [1385.4058837890625, 859.8118896484375], 'packets': [1.4015231132507324, 0.8932565450668335], 'eq_lambda': [1350.97119140625, 858.316162109375], 'avg_pkts_lambda': [0.9117304086685181, 0.9723503589630127], 'exp_max_factor': [6.663637638092041, 4.715115070343018], 'pkts_lambda_on': [0.9116322994232178, 1.651275396347046], 'avg_t_off': [1.6649284362792969, 2.356407403945923], 'avg_t_on': [1.6649284362792969, 2.356407403945923], 'ar_a': [0.0, 1.0], 'sigma': [0.0, 1.0], 'capacity': [27611.091796875, 20090.62109375], 'queue_size': [30259.10546875, 21410.095703125]}
_T = 8
_ITERS = 8


def _lstm(x, h, c, Wx, Wh, b):
    z = x @ Wx + h @ Wh + b
    i, f, g, o = jnp.split(z, 4, axis=-1)
    c = jax.nn.sigmoid(f) * c + jax.nn.sigmoid(i) * jnp.tanh(g)
    h = jax.nn.sigmoid(o) * jnp.tanh(c)
    return h, c


def _rnn(seq, h0, c0, Wx, Wh, b, reverse=False):
    xs = jnp.swapaxes(seq, 0, 1)
    if reverse:
        xs = xs[::-1]
    def step(carry, x):
        h, c = _lstm(x, carry[0], carry[1], Wx, Wh, b)
        return (h, c), h
    (h, c), ys = jax.lax.scan(step, (h0, c0), xs)
    return jnp.swapaxes(ys, 0, 1), h, c


def _mlp2(x, W1, b1, W2, b2):
    return jax.nn.relu(jax.nn.relu(x @ W1 + b1) @ W2 + b2)


_BP = 1000


def _pg(w, hw):
    return jnp.concatenate([w[:, 0:2 * hw], w[:, 3 * hw:4 * hw], w[:, 2 * hw:3 * hw]], axis=1)


def _gates_ifog(z, c, w):
    sig = jax.nn.sigmoid(z[:, 0:3 * w])
    g = jnp.tanh(z[:, 3 * w:4 * w])
    c = sig[:, w:2 * w] * c + sig[:, 0:w] * g
    h = sig[:, 2 * w:3 * w] * jnp.tanh(c)
    return h, c


def _bilstm_body(qg_ref, lg_ref, hf_ref, cf_ref, hb_ref, cb_ref,
                 wqf, wlf, whf, bf, wqb, wlb, whb, bb,
                 pf_ref, pb_ref, hfo, cfo, hbo, cbo):
    hf = hf_ref[...]
    cf = cf_ref[...]
    hb = hb_ref[...]
    cb = cb_ref[...]
    pf_ref[0] = hf
    pb_ref[0] = hb
    for t in range(_T):
        tb = _T - 1 - t
        zf = qg_ref[t] @ wqf[...] + lg_ref[t] @ wlf[...] + bf[...] + hf @ whf[...]
        hf, cf = _gates_ifog(zf, cf, 16)
        zb = qg_ref[tb] @ wqb[...] + lg_ref[tb] @ wlb[...] + bb[...] + hb @ whb[...]
        hb, cb = _gates_ifog(zb, cb, 16)
        pf_ref[1 + t] = hf
        pb_ref[1 + tb] = hb
    hfo[...] = hf
    cfo[...] = cf
    hbo[...] = hb
    cbo[...] = cb


def _tc_bilstm(qg, lg, hf, cf, hb, cb, p):
    n_paths = hf.shape[0]
    grid = n_paths // _BP
    b = _BP
    f32 = jnp.float32
    spec_w = lambda r, c: pl.BlockSpec((r, c), lambda i: (0, 0))
    spec_h = pl.BlockSpec((b, 16), lambda i: (i, 0))
    spec_p = pl.BlockSpec((_T + 1, b, 16), lambda i: (0, i, 0))
    out = pl.pallas_call(
        _bilstm_body,
        grid=(grid,),
        in_specs=[
            pl.BlockSpec((_T, b, 32), lambda i: (0, i, 0)),
            pl.BlockSpec((_T, b, 32), lambda i: (0, i, 0)),
            spec_h, spec_h, spec_h, spec_h,
            spec_w(32, 64), spec_w(32, 64), spec_w(16, 64), spec_w(1, 64),
            spec_w(32, 64), spec_w(32, 64), spec_w(16, 64), spec_w(1, 64),
        ],
        out_specs=[spec_p, spec_p, spec_h, spec_h, spec_h, spec_h],
        out_shape=[
            jax.ShapeDtypeStruct((_T + 1, n_paths, 16), f32),
            jax.ShapeDtypeStruct((_T + 1, n_paths, 16), f32),
            jax.ShapeDtypeStruct((n_paths, 16), f32),
            jax.ShapeDtypeStruct((n_paths, 16), f32),
            jax.ShapeDtypeStruct((n_paths, 16), f32),
            jax.ShapeDtypeStruct((n_paths, 16), f32),
        ],
        compiler_params=pltpu.CompilerParams(
            dimension_semantics=("arbitrary",)),
    )(qg, lg, hf, cf, hb, cb,
      _pg(p['pf_Wx'][:32], 16), _pg(p['pf_Wx'][32:], 16), _pg(p['pf_Wh'], 16), _pg(p['pf_b'][None], 16),
      _pg(p['pb_Wx'][:32], 16), _pg(p['pb_Wx'][32:], 16), _pg(p['pb_Wh'], 16), _pg(p['pb_b'][None], 16))
    return out


def _mlp2_body(x_ref, w1, b1, w2, b2, o_ref):
    h = jax.nn.relu(x_ref[...] @ w1[...] + b1[...])
    o_ref[...] = jax.nn.relu(h @ w2[...] + b2[...])


def _tc_mlp2(x, W1, b1, W2, b2, bp=1000):
    n, di = x.shape
    dh, do = W2.shape
    grid = n // bp
    spec_w = lambda r, c: pl.BlockSpec((r, c), lambda i: (0, 0))
    return pl.pallas_call(
        _mlp2_body,
        grid=(grid,),
        in_specs=[
            pl.BlockSpec((bp, di), lambda i: (i, 0)),
            spec_w(di, dh), spec_w(1, dh), spec_w(dh, do), spec_w(1, do),
        ],
        out_specs=pl.BlockSpec((bp, do), lambda i: (i, 0)),
        out_shape=jax.ShapeDtypeStruct((n, do), jnp.float32),
        compiler_params=pltpu.CompilerParams(dimension_semantics=("arbitrary",)),
    )(x, W1, b1[None], W2, b2[None])


def _qlstm_body(sf_ref, sb_ref, h_ref, c_ref, wxf, wxb, wh, b, ho, co):
    z = (sf_ref[...] @ wxf[...] + sb_ref[...] @ wxb[...]
         + h_ref[...] @ wh[...] + b[...])
    h, c = _gates_ifog(z, c_ref[...], 32)
    ho[...] = h
    co[...] = c


def _tc_queue_lstm(sum_f, sum_b, qh, qc, p, bp=1000):
    n = qh.shape[0]
    grid = n // bp
    spec_w = lambda r, c: pl.BlockSpec((r, c), lambda i: (0, 0))
    spec_h = pl.BlockSpec((bp, 32), lambda i: (i, 0))
    spec_s = pl.BlockSpec((bp, 16), lambda i: (i, 0))
    return pl.pallas_call(
        _qlstm_body,
        grid=(grid,),
        in_specs=[spec_s, spec_s, spec_h, spec_h,
                  spec_w(16, 128), spec_w(16, 128), spec_w(32, 128), spec_w(1, 128)],
        out_specs=[spec_h, spec_h],
        out_shape=[jax.ShapeDtypeStruct((n, 32), jnp.float32)] * 2,
        compiler_params=pltpu.CompilerParams(dimension_semantics=("arbitrary",)),
    )(sum_f, sum_b, qh, qc, _pg(p['qu_Wx'][:16], 32), _pg(p['qu_Wx'][16:], 32),
      _pg(p['qu_Wh'], 32), _pg(p['qu_b'][None], 32))


def _lrnn_body(xg_ref, h_ref, c_ref, wx, wh, b, ho, co):
    h = h_ref[...]
    c = c_ref[...]
    for t in range(3):
        z = xg_ref[t] @ wx[...] + h @ wh[...] + b[...]
        h, c = _gates_ifog(z, c, 32)
    ho[...] = h
    co[...] = c


def _tc_link_rnn(qg2, lh, lc, p, bp=1000):
    n = lh.shape[0]
    grid = n // bp
    spec_w = lambda r, c: pl.BlockSpec((r, c), lambda i: (0, 0))
    spec_h = pl.BlockSpec((bp, 32), lambda i: (i, 0))
    return pl.pallas_call(
        _lrnn_body,
        grid=(grid,),
        in_specs=[pl.BlockSpec((3, bp, 32), lambda i: (0, i, 0)),
                  spec_h, spec_h,
                  spec_w(32, 128), spec_w(32, 128), spec_w(1, 128)],
        out_specs=[spec_h, spec_h],
        out_shape=[jax.ShapeDtypeStruct((n, 32), jnp.float32)] * 2,
        compiler_params=pltpu.CompilerParams(dimension_semantics=("arbitrary",)),
    )(qg2, lh, lc, _pg(p['lu_Wx'], 32), _pg(p['lu_Wh'], 32), _pg(p['lu_b'][None], 32))


def _readout_body(pf_ref, pb_ref, icm_ref, w1, b1, w2, b2, w3, b3, qd_ref, ws_ref):
    qd = jnp.zeros_like(icm_ref[0])
    ws = jnp.zeros_like(qd)
    for t in range(_T):
        x = jnp.concatenate([pf_ref[1 + t], pb_ref[1 + t]], axis=1)
        h1 = jax.nn.relu(x @ w1[...] + b1[...])
        h2 = jax.nn.relu(h1 @ w2[...] + b2[...])
        occ = h2 @ w3[...] + b3[...]
        ic = icm_ref[t]
        qd = qd + occ * ic
        ws = ws + ic
    qd_ref[...] = qd
    ws_ref[...] = ws


def _tc_readout(pss_f, pss_b, icm, p, bp=1000):
    n = pss_f.shape[1]
    grid = n // bp
    spec_w = lambda r, c: pl.BlockSpec((r, c), lambda i: (0, 0))
    spec_p = pl.BlockSpec((_T + 1, bp, 16), lambda i: (0, i, 0))
    spec_i = pl.BlockSpec((_T, bp, 16), lambda i: (0, i, 0))
    spec_o = pl.BlockSpec((bp, 16), lambda i: (i, 0))
    return pl.pallas_call(
        _readout_body,
        grid=(grid,),
        in_specs=[spec_p, spec_p, spec_i,
                  spec_w(32, 16), spec_w(1, 16), spec_w(16, 16), spec_w(1, 16),
                  spec_w(16, 1), spec_w(1, 1)],
        out_specs=[spec_o, spec_o],
        out_shape=[jax.ShapeDtypeStruct((n, 16), jnp.float32)] * 2,
        compiler_params=pltpu.CompilerParams(dimension_semantics=("arbitrary",)),
    )(pss_f, pss_b, icm,
      p['ro_W1'], p['ro_b1'][None], p['ro_W2'], p['ro_b2'][None],
      p['ro_W3'], p['ro_b3'][None])


def _scload_body(tr_hbm, idx_hbm, out_hbm, tr_v, idx_v, out_v, sem):
    c = jax.lax.axis_index("c")
    s = jax.lax.axis_index("s")
    wid = s * 2 + c
    n_l = out_v.shape[0]
    n_g = n_l // 16
    pltpu.sync_copy(tr_hbm, tr_v)
    pltpu.sync_copy(idx_hbm.at[pl.ds(wid * n_l * 64, n_l * 64)], idx_v)

    def body(g, carry):
        acc = jnp.zeros((16,), jnp.float32)
        for j in range(64):
            ix = idx_v[pl.ds(g * 1024 + j * 16, 16)]
            acc = acc + plsc.load_gather(tr_v, [ix])
        out_v[pl.ds(g * 16, 16)] = acc
        return carry

    jax.lax.fori_loop(0, n_g, body, 0)
    pltpu.sync_copy(out_v, out_hbm.at[pl.ds(wid * n_l, n_l)])


def _sc_load(traffic_flat, p2l_grp, n_links_pad):
    n_l = n_links_pad // _NW
    mesh = plsc.VectorSubcoreMesh(core_axis_name="c", subcore_axis_name="s")
    k = functools.partial(
        pl.kernel, mesh=mesh,
        out_type=jax.ShapeDtypeStruct((n_links_pad,), jnp.float32),
        compiler_params=pltpu.CompilerParams(use_tc_tiling_on_sc=False,
                                             needs_layout_passes=False),
        scratch_types=[
            pltpu.VMEM(traffic_flat.shape, jnp.float32),
            pltpu.VMEM((n_l * 64,), jnp.int32),
            pltpu.VMEM((n_l,), jnp.float32),
            pltpu.SemaphoreType.DMA,
        ],
    )(_scload_body)
    return k(traffic_flat, p2l_grp)


def kernel(params, traffic, packets, eq_lambda, avg_pkts_lambda, exp_max_factor, pkts_lambda_on, avg_t_off, avg_t_on, ar_a, sigma, capacity, queue_size, weight, length, model, policy, priority, queue_to_path, link_to_path, path_to_link, path_to_queue, queue_to_link):
    p = params
    nz = lambda v, nm: (v - _ZS[nm][0]) / _ZS[nm][1]
    n_paths = queue_to_path.shape[0]
    n_links = capacity.shape[0]
    n_queues = queue_size.shape[0]
    pkt_size = traffic / packets

    nlp = -(-n_links // (_NW * 16)) * (_NW * 16)
    p2l0 = jnp.pad(path_to_link[:, :, 0].astype(jnp.int32), ((0, nlp - n_links), (0, 0)))
    p2l_grp = jnp.swapaxes(p2l0.reshape(nlp // 16, 16, 64), 1, 2).reshape(-1)
    load = (_sc_load(traffic[:, 0], p2l_grp, nlp)[:n_links, None] / capacity)

    path_feat = jnp.concatenate([nz(traffic, 'traffic'), nz(packets, 'packets'), jax.nn.one_hot(model, 7), nz(eq_lambda, 'eq_lambda'), nz(avg_pkts_lambda, 'avg_pkts_lambda'), nz(exp_max_factor, 'exp_max_factor'), nz(pkts_lambda_on, 'pkts_lambda_on'), nz(avg_t_off, 'avg_t_off'), nz(avg_t_on, 'avg_t_on'), nz(ar_a, 'ar_a'), nz(sigma, 'sigma')], axis=1)
    path_state = _tc_mlp2(path_feat, p['pe_W1'], p['pe_b1'], p['pe_W2'], p['pe_b2'])
    h_f = path_state[:, :16]
    c_f = jnp.zeros_like(h_f)
    h_b = path_state[:, 16:]
    c_b = jnp.zeros_like(h_b)
    lh = _tc_mlp2(jnp.concatenate([load, jax.nn.one_hot(policy, 4)], axis=1), p['le_W1'], p['le_b1'], p['le_W2'], p['le_b2'])
    lc = jnp.zeros_like(lh)
    qh = _tc_mlp2(jnp.concatenate([nz(queue_size, 'queue_size'), jax.nn.one_hot(priority, 3), weight], axis=1), p['qe_W1'], p['qe_b1'], p['qe_W2'], p['qe_b2'])
    qc = jnp.zeros_like(qh)

    q2p_tm = queue_to_path.T.reshape(-1).astype(jnp.int32)
    l2p_tm = link_to_path.T.reshape(-1).astype(jnp.int32)
    q2l_tm = queue_to_link.T.reshape(-1).astype(jnp.int32)
    p2q = path_to_queue.shape[1]
    fidx = (path_to_queue[..., 1] * n_paths + path_to_queue[..., 0]).reshape(-1).astype(jnp.int32)
    fidx_pad = jnp.pad(fidx, (0, (_QPAD - n_queues) * p2q))
    dstq_pad = jnp.repeat(jnp.arange(_QPAD, dtype=jnp.int32) % _QHALF, p2q)
    pss_f = pss_b = None
    for _ in range(_ITERS):
        qg, lg = _sc_gather2(qh, lh, q2p_tm, l2p_tm)
        qg = qg.reshape(_T, n_paths, 32)
        lg = lg.reshape(_T, n_paths, 32)
        pss_f, pss_b, h_f, c_f, h_b, c_b = _tc_bilstm(qg, lg, h_f, c_f, h_b, c_b, p)
        sum_f, sum_b = _sc_gsum2(pss_f.reshape((_T + 1) * n_paths, 16),
                                 pss_b.reshape((_T + 1) * n_paths, 16),
                                 fidx_pad, dstq_pad, n_queues, p2q)
        qh, qc = _tc_queue_lstm(sum_f, sum_b, qh, qc, p)
        qg2 = _sc_gather(qh, q2l_tm).reshape(3, n_links, 32)
        lh, lc = _tc_link_rnn(qg2, lh, lc, p)

    invtab = jnp.tile(jnp.concatenate([1.0 / capacity, jnp.zeros((1, 1), jnp.float32)]), (1, 16))
    invtab8 = jnp.tile(invtab, (8, 1))
    l2p_mask = jnp.where(jnp.arange(_T)[:, None] < length[None, :],
                         link_to_path.T, n_links).reshape(-1).astype(jnp.int32)
    rep = (jnp.arange(l2p_mask.shape[0], dtype=jnp.int32) % 8) * (n_links + 1)
    icm = _sc_gather_rows(invtab8, l2p_mask + rep).reshape(_T, n_paths, 16)
    qd16, ws16 = _tc_readout(pss_f, pss_b, icm, p)
    return qd16[:, :1] + pkt_size * ws16[:, :1]


# speedup vs baseline: 2.2976x; 2.1219x over previous
import functools

import jax
import jax.numpy as jnp
from jax.experimental import pallas as pl
from jax.experimental.pallas import tpu as pltpu
from jax.experimental.pallas import tpu_sc as plsc

_NW = 32


def _sc_gather_rows(table, idx_flat, chunk=1000):
    n = idx_flat.shape[0]
    d = table.shape[1]
    per_w = n // _NW
    n_ch = per_w // chunk
    assert per_w * _NW == n and n_ch * chunk == per_w and chunk % 8 == 0
    mesh = plsc.VectorSubcoreMesh(core_axis_name="c", subcore_axis_name="s")

    @functools.partial(
        pl.kernel, mesh=mesh,
        out_type=jax.ShapeDtypeStruct((n, d), jnp.float32),
        compiler_params=pltpu.CompilerParams(use_tc_tiling_on_sc=False),
        scratch_types=[
            pltpu.VMEM((chunk,), jnp.int32),
            pltpu.VMEM((chunk, d), jnp.float32),
            pltpu.SemaphoreType.DMA,
        ],
    )
    def k(table_hbm, idx_hbm, out_hbm, idx_v, rows_v, sem):
        c = jax.lax.axis_index("c")
        s = jax.lax.axis_index("s")
        wid = s * 2 + c
        base = wid * per_w

        def body(i, carry):
            off = base + i * chunk
            pltpu.sync_copy(idx_hbm.at[pl.ds(off, chunk)], idx_v)
            pltpu.async_copy(table_hbm.at[idx_v], rows_v, sem).wait()
            pltpu.sync_copy(rows_v, out_hbm.at[pl.ds(off, chunk)])
            return carry

        jax.lax.fori_loop(0, n_ch, body, 0)

    return k(table, idx_flat)


def _sc_gather2(qtab, ltab, qidx, lidx, chunk=1000):
    n = qidx.shape[0]
    d = qtab.shape[1]
    per_w = n // _NW
    n_ch = per_w // chunk
    assert per_w * _NW == n and n_ch * chunk == per_w and chunk % 8 == 0
    mesh = plsc.VectorSubcoreMesh(core_axis_name="c", subcore_axis_name="s")

    @functools.partial(
        pl.kernel, mesh=mesh,
        out_type=[jax.ShapeDtypeStruct((n, d), jnp.float32)] * 2,
        compiler_params=pltpu.CompilerParams(use_tc_tiling_on_sc=False),
        scratch_types=[pltpu.VMEM((chunk,), jnp.int32),
                       pltpu.VMEM((chunk,), jnp.int32),
                       pltpu.VMEM((chunk, d), jnp.float32),
                       pltpu.VMEM((chunk, d), jnp.float32),
                       pltpu.SemaphoreType.DMA, pltpu.SemaphoreType.DMA,
                       pltpu.SemaphoreType.DMA, pltpu.SemaphoreType.DMA],
    )
    def k(qt_hbm, lt_hbm, qi_hbm, li_hbm, qo_hbm, lo_hbm,
          qi, li, qr, lr, sgq, sgl, soq, sol):
        c = jax.lax.axis_index("c")
        s = jax.lax.axis_index("s")
        base = (s * 2 + c) * per_w

        def body(j, carry):
            off = base + j * chunk
            pltpu.sync_copy(qi_hbm.at[pl.ds(off, chunk)], qi)
            pltpu.sync_copy(li_hbm.at[pl.ds(off, chunk)], li)
            gq = pltpu.async_copy(qt_hbm.at[qi], qr, sgq)
            gl = pltpu.async_copy(lt_hbm.at[li], lr, sgl)
            gq.wait()
            oq = pltpu.async_copy(qr, qo_hbm.at[pl.ds(off, chunk)], soq)
            gl.wait()
            ol = pltpu.async_copy(lr, lo_hbm.at[pl.ds(off, chunk)], sol)
            oq.wait()
            ol.wait()
            return carry

        jax.lax.fori_loop(0, n_ch, body, 0)

    return k(qtab, ltab, qidx, lidx)


def _pick_chunk(per_w, d, budget_rows=1536):
    best = 8
    for ch in range(8, min(per_w, budget_rows) + 1, 8):
        if per_w % ch == 0:
            best = ch
    return best


def _sc_gather(table, idx_flat):
    n = idx_flat.shape[0]
    n_pad = -(-n // (_NW * 8)) * (_NW * 8)
    if n_pad != n:
        idx_flat = jnp.pad(idx_flat, (0, n_pad - n))
    per_w = n_pad // _NW
    out = _sc_gather_rows(table, idx_flat, chunk=_pick_chunk(per_w, table.shape[1]))
    return out[:n] if n_pad != n else out


_QPW = 288
_QPAD = _QPW * _NW
_QHALF = _QPAD // 2


def _sc_gather_sum(pss_flat, fidx_pad, dstq_pad, n_out, p2q):
    d = pss_flat.shape[1]
    k_edges = p2q
    e_per_w = _QPW * k_edges
    chunk_q = _pick_chunk(_QPW, d, budget_rows=max(8, 1536 // k_edges))
    chunk_e = chunk_q * k_edges
    n_ch = _QPW // chunk_q
    mesh = plsc.VectorSubcoreMesh(core_axis_name="c", subcore_axis_name="s")

    @functools.partial(
        pl.kernel, mesh=mesh,
        out_type=jax.ShapeDtypeStruct((_QPAD, d), jnp.float32),
        compiler_params=pltpu.CompilerParams(use_tc_tiling_on_sc=False),
        scratch_types=[
            pltpu.VMEM((chunk_e,), jnp.int32),
            pltpu.VMEM((chunk_e,), jnp.int32),
            pltpu.VMEM((chunk_e, d), jnp.float32),
            pltpu.VMEM((_QPW, d), jnp.float32),
            pltpu.VMEM_SHARED((_QHALF, d), jnp.float32),
            pltpu.SemaphoreType.DMA,
            pltpu.SemaphoreType.DMA,
        ],
    )
    def k(pss_hbm, fidx_hbm, dstq_hbm, out_hbm, idx_v, dst_v, rows_v, zero_v, acc_sh, sem, sem2):
        c = jax.lax.axis_index("c")
        s = jax.lax.axis_index("s")
        qbase_local = s * _QPW
        qbase_glob = c * _QHALF + s * _QPW
        ebase = qbase_glob * k_edges

        def zbody(i, carry):
            for j0 in range(d // 16):
                zero_v[i, j0 * 16:(j0 + 1) * 16] = jnp.zeros((16,), jnp.float32)
            return carry
        jax.lax.fori_loop(0, _QPW, zbody, 0)
        pltpu.sync_copy(zero_v, acc_sh.at[pl.ds(qbase_local, _QPW)])

        def body(i, carry):
            eoff = ebase + i * chunk_e
            pltpu.sync_copy(fidx_hbm.at[pl.ds(eoff, chunk_e)], idx_v)
            pltpu.sync_copy(dstq_hbm.at[pl.ds(eoff, chunk_e)], dst_v)
            pltpu.async_copy(pss_hbm.at[idx_v], rows_v, sem).wait()
            pltpu.async_copy(rows_v, acc_sh.at[dst_v], sem2, add=True).wait()
            return carry
        jax.lax.fori_loop(0, n_ch, body, 0)

        pltpu.sync_copy(acc_sh.at[pl.ds(qbase_local, _QPW)],
                        out_hbm.at[pl.ds(qbase_glob, _QPW)])

    return k(pss_flat, fidx_pad, dstq_pad)[:n_out]


def _sc_gsum2(pf_flat, pb_flat, fidx_pad, dstq_pad, n_out, p2q):
    d = pf_flat.shape[1]
    k_edges = p2q
    chunk_q = _pick_chunk(_QPW, d, budget_rows=max(8, 1536 // k_edges))
    chunk_e = chunk_q * k_edges
    n_ch = _QPW // chunk_q
    mesh = plsc.VectorSubcoreMesh(core_axis_name="c", subcore_axis_name="s")

    @functools.partial(
        pl.kernel, mesh=mesh,
        out_type=[jax.ShapeDtypeStruct((_QPAD, d), jnp.float32)] * 2,
        compiler_params=pltpu.CompilerParams(use_tc_tiling_on_sc=False),
        scratch_types=[
            pltpu.VMEM((chunk_e,), jnp.int32),
            pltpu.VMEM((chunk_e,), jnp.int32),
            pltpu.VMEM((chunk_e, d), jnp.float32),
            pltpu.VMEM((chunk_e, d), jnp.float32),
            pltpu.VMEM((_QPW, d), jnp.float32),
            pltpu.VMEM_SHARED((_QHALF, d), jnp.float32),
            pltpu.VMEM_SHARED((_QHALF, d), jnp.float32),
            pltpu.SemaphoreType.DMA, pltpu.SemaphoreType.DMA,
            pltpu.SemaphoreType.DMA, pltpu.SemaphoreType.DMA,
        ],
    )
    def k(pf_hbm, pb_hbm, fidx_hbm, dstq_hbm, of_hbm, ob_hbm,
          idx_v, dst_v, rf_v, rb_v, zero_v, accf, accb, sgf, sgb, saf, sab):
        c = jax.lax.axis_index("c")
        s = jax.lax.axis_index("s")
        qbase_local = s * _QPW
        qbase_glob = c * _QHALF + s * _QPW
        ebase = qbase_glob * k_edges

        def zbody(i, carry):
            for j0 in range(d // 16):
                zero_v[i, j0 * 16:(j0 + 1) * 16] = jnp.zeros((16,), jnp.float32)
            return carry
        jax.lax.fori_loop(0, _QPW, zbody, 0)
        pltpu.sync_copy(zero_v, accf.at[pl.ds(qbase_local, _QPW)])
        pltpu.sync_copy(zero_v, accb.at[pl.ds(qbase_local, _QPW)])

        def body(i, carry):
            eoff = ebase + i * chunk_e
            pltpu.sync_copy(fidx_hbm.at[pl.ds(eoff, chunk_e)], idx_v)
            pltpu.sync_copy(dstq_hbm.at[pl.ds(eoff, chunk_e)], dst_v)
            gf = pltpu.async_copy(pf_hbm.at[idx_v], rf_v, sgf)
            gb = pltpu.async_copy(pb_hbm.at[idx_v], rb_v, sgb)
            gf.wait()
            af = pltpu.async_copy(rf_v, accf.at[dst_v], saf, add=True)
            gb.wait()
            ab = pltpu.async_copy(rb_v, accb.at[dst_v], sab, add=True)
            af.wait()
            ab.wait()
            return carry
        jax.lax.fori_loop(0, n_ch, body, 0)

        pltpu.sync_copy(accf.at[pl.ds(qbase_local, _QPW)],
                        of_hbm.at[pl.ds(qbase_glob, _QPW)])
        pltpu.sync_copy(accb.at[pl.ds(qbase_local, _QPW)],
                        ob_hbm.at[pl.ds(qbase_glob, _QPW)])

    of, ob = k(pf_flat, pb_flat, fidx_pad, dstq_pad)
    return of[:n_out], ob[:n_out]

_ZS = {'traffic': [1385.4058837890625, 859.8118896484375], 'packets': [1.4015231132507324, 0.8932565450668335], 'eq_lambda': [1350.97119140625, 858.316162109375], 'avg_pkts_lambda': [0.9117304086685181, 0.9723503589630127], 'exp_max_factor': [6.663637638092041, 4.715115070343018], 'pkts_lambda_on': [0.9116322994232178, 1.651275396347046], 'avg_t_off': [1.6649284362792969, 2.356407403945923], 'avg_t_on': [1.6649284362792969, 2.356407403945923], 'ar_a': [0.0, 1.0], 'sigma': [0.0, 1.0], 'capacity': [27611.091796875, 20090.62109375], 'queue_size': [30259.10546875, 21410.095703125]}
_T = 8
_ITERS = 8


def _lstm(x, h, c, Wx, Wh, b):
    z = x @ Wx + h @ Wh + b
    i, f, g, o = jnp.split(z, 4, axis=-1)
    c = jax.nn.sigmoid(f) * c + jax.nn.sigmoid(i) * jnp.tanh(g)
    h = jax.nn.sigmoid(o) * jnp.tanh(c)
    return h, c


def _rnn(seq, h0, c0, Wx, Wh, b, reverse=False):
    xs = jnp.swapaxes(seq, 0, 1)
    if reverse:
        xs = xs[::-1]
    def step(carry, x):
        h, c = _lstm(x, carry[0], carry[1], Wx, Wh, b)
        return (h, c), h
    (h, c), ys = jax.lax.scan(step, (h0, c0), xs)
    return jnp.swapaxes(ys, 0, 1), h, c


def _mlp2(x, W1, b1, W2, b2):
    return jax.nn.relu(jax.nn.relu(x @ W1 + b1) @ W2 + b2)


_BP = 4000


def _pg(w, hw):
    return jnp.concatenate([w[:, 0:2 * hw], w[:, 3 * hw:4 * hw], w[:, 2 * hw:3 * hw]], axis=1)


def _gates_ifog(z, c, w):
    sig = jax.nn.sigmoid(z[:, 0:3 * w])
    g = jnp.tanh(z[:, 3 * w:4 * w])
    c = sig[:, w:2 * w] * c + sig[:, 0:w] * g
    h = sig[:, 2 * w:3 * w] * jnp.tanh(c)
    return h, c


def _bilstm_body(qg_ref, lg_ref, hf_ref, cf_ref, hb_ref, cb_ref,
                 wqf, wlf, whf, bf, wqb, wlb, whb, bb,
                 pf_ref, pb_ref, hfo, cfo, hbo, cbo):
    hf = hf_ref[...]
    cf = cf_ref[...]
    hb = hb_ref[...]
    cb = cb_ref[...]
    pf_ref[0] = hf
    pb_ref[0] = hb
    for t in range(_T):
        tb = _T - 1 - t
        zf = qg_ref[t] @ wqf[...] + lg_ref[t] @ wlf[...] + bf[...] + hf @ whf[...]
        hf, cf = _gates_ifog(zf, cf, 64)
        zb = qg_ref[tb] @ wqb[...] + lg_ref[tb] @ wlb[...] + bb[...] + hb @ whb[...]
        hb, cb = _gates_ifog(zb, cb, 64)
        pf_ref[1 + t] = hf
        pb_ref[1 + tb] = hb
    hfo[...] = hf
    cfo[...] = cf
    hbo[...] = hb
    cbo[...] = cb


def _kron4_x(w):
    wp = _pg(w, 16).reshape(32, 4, 16)
    return jnp.einsum('jk,vgu->jvgku', jnp.eye(4, dtype=jnp.float32), wp).reshape(128, 256)


def _kron4_h(w):
    wp = _pg(w, 16).reshape(16, 4, 16)
    return jnp.einsum('jk,ugw->jugkw', jnp.eye(4, dtype=jnp.float32), wp).reshape(64, 256)


def _kron4_b(b):
    bp = _pg(b[None], 16).reshape(4, 16)
    return jnp.tile(bp[:, None, :], (1, 4, 1)).reshape(1, 256)


def _tc_bilstm(qg_pk, lg_pk, hf, cf, hb, cb, p):
    npk = hf.shape[0]
    b = _BP // 4
    grid = npk // b
    f32 = jnp.float32
    spec_w = lambda r, c: pl.BlockSpec((r, c), lambda i: (0, 0))
    spec_h = pl.BlockSpec((b, 64), lambda i: (i, 0))
    spec_p = pl.BlockSpec((_T + 1, b, 64), lambda i: (0, i, 0))
    out = pl.pallas_call(
        _bilstm_body,
        grid=(grid,),
        in_specs=[
            pl.BlockSpec((_T, b, 128), lambda i: (0, i, 0)),
            pl.BlockSpec((_T, b, 128), lambda i: (0, i, 0)),
            spec_h, spec_h, spec_h, spec_h,
            spec_w(128, 256), spec_w(128, 256), spec_w(64, 256), spec_w(1, 256),
            spec_w(128, 256), spec_w(128, 256), spec_w(64, 256), spec_w(1, 256),
        ],
        out_specs=[spec_p, spec_p, spec_h, spec_h, spec_h, spec_h],
        out_shape=[
            jax.ShapeDtypeStruct((_T + 1, npk, 64), f32),
            jax.ShapeDtypeStruct((_T + 1, npk, 64), f32),
            jax.ShapeDtypeStruct((npk, 64), f32),
            jax.ShapeDtypeStruct((npk, 64), f32),
            jax.ShapeDtypeStruct((npk, 64), f32),
            jax.ShapeDtypeStruct((npk, 64), f32),
        ],
        compiler_params=pltpu.CompilerParams(
            dimension_semantics=("arbitrary",)),
    )(qg_pk, lg_pk, hf, cf, hb, cb,
      _kron4_x(p['pf_Wx'][:32]), _kron4_x(p['pf_Wx'][32:]), _kron4_h(p['pf_Wh']), _kron4_b(p['pf_b']),
      _kron4_x(p['pb_Wx'][:32]), _kron4_x(p['pb_Wx'][32:]), _kron4_h(p['pb_Wh']), _kron4_b(p['pb_b']))
    return out


def _mlp2_body(x_ref, w1, b1, w2, b2, o_ref):
    h = jax.nn.relu(x_ref[...] @ w1[...] + b1[...])
    o_ref[...] = jax.nn.relu(h @ w2[...] + b2[...])


def _tc_mlp2(x, W1, b1, W2, b2, bp=1000):
    n, di = x.shape
    dh, do = W2.shape
    grid = n // bp
    spec_w = lambda r, c: pl.BlockSpec((r, c), lambda i: (0, 0))
    return pl.pallas_call(
        _mlp2_body,
        grid=(grid,),
        in_specs=[
            pl.BlockSpec((bp, di), lambda i: (i, 0)),
            spec_w(di, dh), spec_w(1, dh), spec_w(dh, do), spec_w(1, do),
        ],
        out_specs=pl.BlockSpec((bp, do), lambda i: (i, 0)),
        out_shape=jax.ShapeDtypeStruct((n, do), jnp.float32),
        compiler_params=pltpu.CompilerParams(dimension_semantics=("arbitrary",)),
    )(x, W1, b1[None], W2, b2[None])


def _qlstm_body(sf_ref, sb_ref, h_ref, c_ref, wxf, wxb, wh, b, ho, co):
    z = (sf_ref[...] @ wxf[...] + sb_ref[...] @ wxb[...]
         + h_ref[...] @ wh[...] + b[...])
    h, c = _gates_ifog(z, c_ref[...], 32)
    ho[...] = h
    co[...] = c


def _tc_queue_lstm(sum_f, sum_b, qh, qc, p, bp=1000):
    n = qh.shape[0]
    grid = n // bp
    spec_w = lambda r, c: pl.BlockSpec((r, c), lambda i: (0, 0))
    spec_h = pl.BlockSpec((bp, 32), lambda i: (i, 0))
    spec_s = pl.BlockSpec((bp, 16), lambda i: (i, 0))
    return pl.pallas_call(
        _qlstm_body,
        grid=(grid,),
        in_specs=[spec_s, spec_s, spec_h, spec_h,
                  spec_w(16, 128), spec_w(16, 128), spec_w(32, 128), spec_w(1, 128)],
        out_specs=[spec_h, spec_h],
        out_shape=[jax.ShapeDtypeStruct((n, 32), jnp.float32)] * 2,
        compiler_params=pltpu.CompilerParams(dimension_semantics=("arbitrary",)),
    )(sum_f, sum_b, qh, qc, _pg(p['qu_Wx'][:16], 32), _pg(p['qu_Wx'][16:], 32),
      _pg(p['qu_Wh'], 32), _pg(p['qu_b'][None], 32))


def _lrnn_body(xg_ref, h_ref, c_ref, wx, wh, b, ho, co):
    h = h_ref[...]
    c = c_ref[...]
    for t in range(3):
        z = xg_ref[t] @ wx[...] + h @ wh[...] + b[...]
        h, c = _gates_ifog(z, c, 32)
    ho[...] = h
    co[...] = c


def _tc_link_rnn(qg2, lh, lc, p, bp=1000):
    n = lh.shape[0]
    grid = n // bp
    spec_w = lambda r, c: pl.BlockSpec((r, c), lambda i: (0, 0))
    spec_h = pl.BlockSpec((bp, 32), lambda i: (i, 0))
    return pl.pallas_call(
        _lrnn_body,
        grid=(grid,),
        in_specs=[pl.BlockSpec((3, bp, 32), lambda i: (0, i, 0)),
                  spec_h, spec_h,
                  spec_w(32, 128), spec_w(32, 128), spec_w(1, 128)],
        out_specs=[spec_h, spec_h],
        out_shape=[jax.ShapeDtypeStruct((n, 32), jnp.float32)] * 2,
        compiler_params=pltpu.CompilerParams(dimension_semantics=("arbitrary",)),
    )(qg2, lh, lc, _pg(p['lu_Wx'], 32), _pg(p['lu_Wh'], 32), _pg(p['lu_b'][None], 32))


def _readout_body(pf_ref, pb_ref, icm_ref, w1, b1, w2, b2, w3, b3, qd_ref, ws_ref):
    qd = jnp.zeros_like(icm_ref[0])
    ws = jnp.zeros_like(qd)
    for t in range(_T):
        x = jnp.concatenate([pf_ref[1 + t], pb_ref[1 + t]], axis=1)
        h1 = jax.nn.relu(x @ w1[...] + b1[...])
        h2 = jax.nn.relu(h1 @ w2[...] + b2[...])
        occ = h2 @ w3[...] + b3[...]
        ic = icm_ref[t]
        qd = qd + occ * ic
        ws = ws + ic
    qd_ref[...] = qd
    ws_ref[...] = ws


def _tc_readout(pss_f, pss_b, icm, p, bp=1000):
    n = pss_f.shape[1]
    grid = n // bp
    spec_w = lambda r, c: pl.BlockSpec((r, c), lambda i: (0, 0))
    spec_p = pl.BlockSpec((_T + 1, bp, 16), lambda i: (0, i, 0))
    spec_i = pl.BlockSpec((_T, bp, 16), lambda i: (0, i, 0))
    spec_o = pl.BlockSpec((bp, 16), lambda i: (i, 0))
    return pl.pallas_call(
        _readout_body,
        grid=(grid,),
        in_specs=[spec_p, spec_p, spec_i,
                  spec_w(32, 16), spec_w(1, 16), spec_w(16, 16), spec_w(1, 16),
                  spec_w(16, 1), spec_w(1, 1)],
        out_specs=[spec_o, spec_o],
        out_shape=[jax.ShapeDtypeStruct((n, 16), jnp.float32)] * 2,
        compiler_params=pltpu.CompilerParams(dimension_semantics=("arbitrary",)),
    )(pss_f, pss_b, icm,
      p['ro_W1'], p['ro_b1'][None], p['ro_W2'], p['ro_b2'][None],
      p['ro_W3'], p['ro_b3'][None])


def _scload_body(tr_hbm, idx_hbm, out_hbm, tr_v, idx_v, out_v, sem):
    c = jax.lax.axis_index("c")
    s = jax.lax.axis_index("s")
    wid = s * 2 + c
    n_l = out_v.shape[0]
    n_g = n_l // 16
    pltpu.sync_copy(tr_hbm, tr_v)
    pltpu.sync_copy(idx_hbm.at[pl.ds(wid * n_l * 64, n_l * 64)], idx_v)

    def body(g, carry):
        acc = jnp.zeros((16,), jnp.float32)
        for j in range(64):
            ix = idx_v[pl.ds(g * 1024 + j * 16, 16)]
            acc = acc + plsc.load_gather(tr_v, [ix])
        out_v[pl.ds(g * 16, 16)] = acc
        return carry

    jax.lax.fori_loop(0, n_g, body, 0)
    pltpu.sync_copy(out_v, out_hbm.at[pl.ds(wid * n_l, n_l)])


def _sc_load(traffic_flat, p2l_grp, n_links_pad):
    n_l = n_links_pad // _NW
    mesh = plsc.VectorSubcoreMesh(core_axis_name="c", subcore_axis_name="s")
    k = functools.partial(
        pl.kernel, mesh=mesh,
        out_type=jax.ShapeDtypeStruct((n_links_pad,), jnp.float32),
        compiler_params=pltpu.CompilerParams(use_tc_tiling_on_sc=False,
                                             needs_layout_passes=False),
        scratch_types=[
            pltpu.VMEM(traffic_flat.shape, jnp.float32),
            pltpu.VMEM((n_l * 64,), jnp.int32),
            pltpu.VMEM((n_l,), jnp.float32),
            pltpu.SemaphoreType.DMA,
        ],
    )(_scload_body)
    return k(traffic_flat, p2l_grp)


def kernel(params, traffic, packets, eq_lambda, avg_pkts_lambda, exp_max_factor, pkts_lambda_on, avg_t_off, avg_t_on, ar_a, sigma, capacity, queue_size, weight, length, model, policy, priority, queue_to_path, link_to_path, path_to_link, path_to_queue, queue_to_link):
    p = params
    nz = lambda v, nm: (v - _ZS[nm][0]) / _ZS[nm][1]
    n_paths = queue_to_path.shape[0]
    n_links = capacity.shape[0]
    n_queues = queue_size.shape[0]
    pkt_size = traffic / packets

    nlp = -(-n_links // (_NW * 16)) * (_NW * 16)
    p2l0 = jnp.pad(path_to_link[:, :, 0].astype(jnp.int32), ((0, nlp - n_links), (0, 0)))
    p2l_grp = jnp.swapaxes(p2l0.reshape(nlp // 16, 16, 64), 1, 2).reshape(-1)
    load = (_sc_load(traffic[:, 0], p2l_grp, nlp)[:n_links, None] / capacity)

    path_feat = jnp.concatenate([nz(traffic, 'traffic'), nz(packets, 'packets'), jax.nn.one_hot(model, 7), nz(eq_lambda, 'eq_lambda'), nz(avg_pkts_lambda, 'avg_pkts_lambda'), nz(exp_max_factor, 'exp_max_factor'), nz(pkts_lambda_on, 'pkts_lambda_on'), nz(avg_t_off, 'avg_t_off'), nz(avg_t_on, 'avg_t_on'), nz(ar_a, 'ar_a'), nz(sigma, 'sigma')], axis=1)
    path_state = _tc_mlp2(path_feat, p['pe_W1'], p['pe_b1'], p['pe_W2'], p['pe_b2'])
    h_f = path_state[:, :16].reshape(n_paths // 4, 64)
    c_f = jnp.zeros_like(h_f)
    h_b = path_state[:, 16:].reshape(n_paths // 4, 64)
    c_b = jnp.zeros_like(h_b)
    lh = _tc_mlp2(jnp.concatenate([load, jax.nn.one_hot(policy, 4)], axis=1), p['le_W1'], p['le_b1'], p['le_W2'], p['le_b2'])
    lc = jnp.zeros_like(lh)
    qh = _tc_mlp2(jnp.concatenate([nz(queue_size, 'queue_size'), jax.nn.one_hot(priority, 3), weight], axis=1), p['qe_W1'], p['qe_b1'], p['qe_W2'], p['qe_b2'])
    qc = jnp.zeros_like(qh)

    q2p_tm = queue_to_path.T.reshape(-1).astype(jnp.int32)
    l2p_tm = link_to_path.T.reshape(-1).astype(jnp.int32)
    q2l_tm = queue_to_link.T.reshape(-1).astype(jnp.int32)
    p2q = path_to_queue.shape[1]
    fidx = (path_to_queue[..., 1] * n_paths + path_to_queue[..., 0]).reshape(-1).astype(jnp.int32)
    fidx_pad = jnp.pad(fidx, (0, (_QPAD - n_queues) * p2q))
    dstq_pad = jnp.repeat(jnp.arange(_QPAD, dtype=jnp.int32) % _QHALF, p2q)
    pss_f = pss_b = None
    for _ in range(_ITERS):
        qg, lg = _sc_gather2(qh, lh, q2p_tm, l2p_tm)
        qg = qg.reshape(_T, n_paths // 4, 128)
        lg = lg.reshape(_T, n_paths // 4, 128)
        pss_f, pss_b, h_f, c_f, h_b, c_b = _tc_bilstm(qg, lg, h_f, c_f, h_b, c_b, p)
        sum_f, sum_b = _sc_gsum2(pss_f.reshape((_T + 1) * n_paths, 16),
                                 pss_b.reshape((_T + 1) * n_paths, 16),
                                 fidx_pad, dstq_pad, n_queues, p2q)
        qh, qc = _tc_queue_lstm(sum_f, sum_b, qh, qc, p)
        qg2 = _sc_gather(qh, q2l_tm).reshape(3, n_links, 32)
        lh, lc = _tc_link_rnn(qg2, lh, lc, p)

    invtab = jnp.tile(jnp.concatenate([1.0 / capacity, jnp.zeros((1, 1), jnp.float32)]), (1, 16))
    invtab8 = jnp.tile(invtab, (8, 1))
    l2p_mask = jnp.where(jnp.arange(_T)[:, None] < length[None, :],
                         link_to_path.T, n_links).reshape(-1).astype(jnp.int32)
    rep = (jnp.arange(l2p_mask.shape[0], dtype=jnp.int32) % 8) * (n_links + 1)
    icm = _sc_gather_rows(invtab8, l2p_mask + rep).reshape(_T, n_paths, 16)
    qd16, ws16 = _tc_readout(pss_f.reshape(_T + 1, n_paths, 16),
                             pss_b.reshape(_T + 1, n_paths, 16), icm, p)
    return qd16[:, :1] + pkt_size * ws16[:, :1]
